# Initial kernel scaffold; baseline (speedup 1.0000x reference)
import jax
import jax.numpy as jnp
from jax import lax
from jax.experimental import pallas as pl
from jax.experimental.pallas import tpu as pltpu
from jax.experimental.pallas import tpu_sc as plsc

NN = 10000
NE = 320000
HID = 128
HH = 64
NTRIP = 32768
F32 = jnp.float32

_NC, _NS = 2, 16
_CHUNK = 128
_EPT = NE // _NS
_NFULL = _EPT // _CHUNK
_REM = _EPT - _NFULL * _CHUNK
_NPT = NN // _NS


def _sc_mesh():
    return plsc.VectorSubcoreMesh(
        core_axis_name="c", subcore_axis_name="s",
        num_cores=_NC, num_subcores=_NS)


def _make_edge_pass(write_eraw: bool):
    outs = [
        jax.ShapeDtypeStruct((_NC * NN, HH), F32),
        jax.ShapeDtypeStruct((_NC * NN, HH), F32),
    ]
    if write_eraw:
        outs.append(jax.ShapeDtypeStruct((_NC * NE, HH), F32))
    scratch = [
        pltpu.VMEM((_CHUNK,), jnp.int32),
        pltpu.VMEM((_CHUNK,), jnp.int32),
        pltpu.VMEM((_CHUNK,), jnp.int32),
        pltpu.VMEM((_CHUNK, HH), F32),
        pltpu.VMEM((_CHUNK, HID), F32),
        pltpu.VMEM((_CHUNK, HH), F32),
        pltpu.VMEM((_CHUNK, HH), F32),
        pltpu.VMEM_SHARED((NN, HH), F32),
        pltpu.VMEM_SHARED((NN, HH), F32),
    ]

    def body(ce_hbm, db_hbm, eh_hbm, src_hbm, dst_hbm, *rest):
        if write_eraw:
            (num_hbm, den_hbm, eraw_hbm,
             srcv, dstv, dst2v, cev, dbv, ehv, sigv, num_sp, den_sp) = rest
        else:
            (num_hbm, den_hbm,
             srcv, dstv, dst2v, cev, dbv, ehv, sigv, num_sp, den_sp) = rest
            eraw_hbm = None

        cid = lax.axis_index("c")
        sid = lax.axis_index("s")
        zero16 = jnp.zeros((16,), F32)
        row_off = cid * NN
        ce_off = cid * NE

        def zrow(r, _):
            for v in range(HH // 16):
                sigv[r, pl.ds(v * 16, 16)] = zero16
            return 0
        lax.fori_loop(0, _CHUNK, zrow, 0)
        zbase = sid * _NPT
        off = 0
        for nr in (128, 128, 128, 128, _NPT - 4 * 128):
            pltpu.sync_copy(sigv.at[pl.ds(0, nr)], num_sp.at[pl.ds(zbase + off, nr)])
            pltpu.sync_copy(sigv.at[pl.ds(0, nr)], den_sp.at[pl.ds(zbase + off, nr)])
            off += nr
        plsc.subcore_barrier()

        tbase = sid * _EPT

        def chunk(ebase, nrows):
            pltpu.sync_copy(src_hbm.at[pl.ds(ebase, nrows)], srcv.at[pl.ds(0, nrows)])
            pltpu.sync_copy(dst_hbm.at[pl.ds(ebase, nrows)], dstv.at[pl.ds(0, nrows)])
            off_vec = jnp.zeros((16,), jnp.int32) + row_off
            for v in range(_CHUNK // 16):
                sl = pl.ds(v * 16, 16)
                if v * 16 < nrows:
                    srcv[sl] = srcv[sl] + row_off
                    dst2v[sl] = dstv[sl] + row_off
                else:
                    srcv[sl] = off_vec
                    dst2v[sl] = off_vec
            pltpu.sync_copy(ce_hbm.at[pl.ds(ce_off + ebase, nrows)],
                            cev.at[pl.ds(0, nrows)])
            pltpu.sync_copy(db_hbm.at[srcv], dbv)
            pltpu.sync_copy(eh_hbm.at[dst2v], ehv)

            def crow(r, _):
                for v in range(HH // 16):
                    sl = pl.ds(v * 16, 16)
                    e = cev[r, sl] + dbv[r, sl] + ehv[r, sl]
                    cev[r, sl] = e
                    s = 1.0 / (1.0 + jnp.exp(-e))
                    sigv[r, sl] = s
                    ehv[r, sl] = s * dbv[r, pl.ds(HH + v * 16, 16)]
                return 0
            lax.fori_loop(0, nrows, crow, 0)
            if nrows < _CHUNK:
                def zrow2(r, _):
                    for v in range(HH // 16):
                        sl = pl.ds(v * 16, 16)
                        sigv[r, sl] = zero16
                        ehv[r, sl] = zero16
                    return 0
                lax.fori_loop(nrows, _CHUNK, zrow2, 0)
            if eraw_hbm is not None:
                pltpu.sync_copy(cev.at[pl.ds(0, nrows)],
                                eraw_hbm.at[pl.ds(ce_off + ebase, nrows)])
            pltpu.sync_copy(ehv, num_sp.at[dstv], add=True)
            pltpu.sync_copy(sigv, den_sp.at[dstv], add=True)

        def loop_body(c, _):
            chunk(tbase + c * _CHUNK, _CHUNK)
            return 0
        lax.fori_loop(0, _NFULL, loop_body, 0)
        chunk(tbase + _NFULL * _CHUNK, _REM)

        plsc.subcore_barrier()
        fbase = sid * _NPT
        pltpu.sync_copy(num_sp.at[pl.ds(fbase, _NPT)],
                        num_hbm.at[pl.ds(row_off + fbase, _NPT)])
        pltpu.sync_copy(den_sp.at[pl.ds(fbase, _NPT)],
                        den_hbm.at[pl.ds(row_off + fbase, _NPT)])

    return pl.kernel(body, out_type=tuple(outs), mesh=_sc_mesh(),
                     scratch_types=scratch)


_edge_pass_w = _make_edge_pass(True)
_edge_pass_nw = _make_edge_pass(False)


_TPW = NTRIP // (_NC * _NS)
_TCH = _TPW // _CHUNK


def _make_head_gather():
    outs = (jax.ShapeDtypeStruct((NTRIP, HID), F32),
            jax.ShapeDtypeStruct((NTRIP, HID), F32))
    scratch = [
        pltpu.VMEM((_CHUNK,), jnp.int32),
        pltpu.VMEM((_CHUNK, HID), F32),
    ]

    def body(h_hbm, t0_hbm, t2_hbm, s_hbm, o_hbm, idxv, rowsv):
        cid = lax.axis_index("c")
        sid = lax.axis_index("s")
        base = (sid * _NC + cid) * _TPW
        for t_hbm, out_hbm in ((t0_hbm, s_hbm), (t2_hbm, o_hbm)):
            def cb(c, _, t_hbm=t_hbm, out_hbm=out_hbm):
                rb = base + c * _CHUNK
                pltpu.sync_copy(t_hbm.at[pl.ds(rb, _CHUNK)], idxv)
                pltpu.sync_copy(h_hbm.at[idxv], rowsv)
                pltpu.sync_copy(rowsv, out_hbm.at[pl.ds(rb, _CHUNK)])
                return 0
            lax.fori_loop(0, _TCH, cb, 0)

    return pl.kernel(body, out_type=outs, mesh=_sc_mesh(),
                     scratch_types=scratch)


_head_gather = _make_head_gather()


def _dot(a, b):
    return jnp.dot(a, b, preferred_element_type=F32)


def _proj_node(x, w, b):
    def body(x_ref, w_ref, b_ref, o_ref):
        o_ref[...] = _dot(x_ref[...], w_ref[...]) + b_ref[...]
    return pl.pallas_call(
        body, out_shape=jax.ShapeDtypeStruct((NN, HID), F32),
    )(x, w, b.reshape(1, HID))


_ER = 3200
_EG = NE // _ER


def _proj_edge(et, lw, lb, cw, cb):
    def body(x_ref, lw_ref, lb_ref, cw_ref, cb_ref, e_ref, ce_ref):
        e = _dot(x_ref[...], lw_ref[...]) + lb_ref[...]
        e_ref[0] = e[:, :HH]
        e_ref[1] = e[:, HH:]
        ce = _dot(e, cw_ref[...]) + cb_ref[...]
        ce_ref[0] = ce[:, :HH]
        ce_ref[1] = ce[:, HH:]
    return pl.pallas_call(
        body,
        grid=(_EG,),
        in_specs=[
            pl.BlockSpec((_ER, 16), lambda i: (i, 0)),
            pl.BlockSpec((16, HID), lambda i: (0, 0)),
            pl.BlockSpec((1, HID), lambda i: (0, 0)),
            pl.BlockSpec((HID, HID), lambda i: (0, 0)),
            pl.BlockSpec((1, HID), lambda i: (0, 0)),
        ],
        out_specs=[
            pl.BlockSpec((2, _ER, HH), lambda i: (0, i, 0)),
            pl.BlockSpec((2, _ER, HH), lambda i: (0, i, 0)),
        ],
        out_shape=[
            jax.ShapeDtypeStruct((2, NE, HH), F32),
            jax.ShapeDtypeStruct((2, NE, HH), F32),
        ],
    )(et, lw, lb.reshape(1, HID), cw, cb.reshape(1, HID))


def _node_mm(h, lp):
    def body(h_ref, aw, ab, bw, bb, dw, dbias, ew, eb, ah_ref, dbt_ref, eht_ref):
        hh = h_ref[...]
        Ah = _dot(hh, aw[...]) + ab[...]
        Bh = _dot(hh, bw[...]) + bb[...]
        Dh = _dot(hh, dw[...]) + dbias[...]
        Eh = _dot(hh, ew[...]) + eb[...]
        ah_ref[...] = Ah
        dbt_ref[0] = jnp.concatenate([Dh[:, :HH], Bh[:, :HH]], axis=1)
        dbt_ref[1] = jnp.concatenate([Dh[:, HH:], Bh[:, HH:]], axis=1)
        eht_ref[0] = Eh[:, :HH]
        eht_ref[1] = Eh[:, HH:]
    return pl.pallas_call(
        body,
        out_shape=[
            jax.ShapeDtypeStruct((NN, HID), F32),
            jax.ShapeDtypeStruct((2, NN, HID), F32),
            jax.ShapeDtypeStruct((2, NN, HH), F32),
        ],
    )(h, lp['A_w'], lp['A_b'].reshape(1, HID),
      lp['B_w'], lp['B_b'].reshape(1, HID),
      lp['D_w'], lp['D_b'].reshape(1, HID),
      lp['E_w'], lp['E_b'].reshape(1, HID))


def _h_update(h_in, ah, num, den, norm_n, bn_g, bn_b):
    def body(hin_ref, ah_ref, num_ref, den_ref, nn_ref, g_ref, b_ref, out_ref):
        num_f = jnp.concatenate([num_ref[0], num_ref[1]], axis=1)
        den_f = jnp.concatenate([den_ref[0], den_ref[1]], axis=1)
        hn = ah_ref[...] + num_f / (den_f + 1e-6)
        hn = hn * nn_ref[...]
        mu = jnp.mean(hn, axis=0, keepdims=True)
        var = jnp.mean((hn - mu) ** 2, axis=0, keepdims=True)
        hn = (hn - mu) * lax.rsqrt(var + 1e-5) * g_ref[...] + b_ref[...]
        out_ref[...] = hin_ref[...] + jnp.maximum(hn, 0.0)
    return pl.pallas_call(
        body, out_shape=jax.ShapeDtypeStruct((NN, HID), F32),
    )(h_in, ah, num, den, norm_n, bn_g.reshape(1, HID), bn_b.reshape(1, HID))


def _e_stats(eraw, norm_e):
    def body(er_ref, ne_ref, out_ref):
        i = pl.program_id(0)
        y = jnp.concatenate([er_ref[0], er_ref[1]], axis=1) * ne_ref[...]
        @pl.when(i == 0)
        def _():
            out_ref[...] = jnp.zeros_like(out_ref)
        out_ref[0:1, :] += jnp.sum(y, axis=0, keepdims=True)
        out_ref[1:2, :] += jnp.sum(y * y, axis=0, keepdims=True)
    return pl.pallas_call(
        body,
        grid=(_EG,),
        in_specs=[
            pl.BlockSpec((2, _ER, HH), lambda i: (0, i, 0)),
            pl.BlockSpec((_ER, 1), lambda i: (i, 0)),
        ],
        out_specs=pl.BlockSpec((8, HID), lambda i: (0, 0)),
        out_shape=jax.ShapeDtypeStruct((8, HID), F32),
    )(eraw, norm_e)


def _e_apply(eraw, e_in, norm_e, st, bn_g, bn_b, cw, cb):
    def body(er_ref, ein_ref, ne_ref, st_ref, g_ref, b_ref, cw_ref, cb_ref,
             enew_ref, ce_ref):
        y = jnp.concatenate([er_ref[0], er_ref[1]], axis=1) * ne_ref[...]
        mu = st_ref[0:1, :] * (1.0 / NE)
        var = st_ref[1:2, :] * (1.0 / NE) - mu * mu
        z = (y - mu) * lax.rsqrt(var + 1e-5) * g_ref[...] + b_ref[...]
        z = jnp.maximum(z, 0.0)
        enew = jnp.concatenate([ein_ref[0], ein_ref[1]], axis=1) + z
        enew_ref[0] = enew[:, :HH]
        enew_ref[1] = enew[:, HH:]
        ce = _dot(enew, cw_ref[...]) + cb_ref[...]
        ce_ref[0] = ce[:, :HH]
        ce_ref[1] = ce[:, HH:]
    return pl.pallas_call(
        body,
        grid=(_EG,),
        in_specs=[
            pl.BlockSpec((2, _ER, HH), lambda i: (0, i, 0)),
            pl.BlockSpec((2, _ER, HH), lambda i: (0, i, 0)),
            pl.BlockSpec((_ER, 1), lambda i: (i, 0)),
            pl.BlockSpec((8, HID), lambda i: (0, 0)),
            pl.BlockSpec((1, HID), lambda i: (0, 0)),
            pl.BlockSpec((1, HID), lambda i: (0, 0)),
            pl.BlockSpec((HID, HID), lambda i: (0, 0)),
            pl.BlockSpec((1, HID), lambda i: (0, 0)),
        ],
        out_specs=[
            pl.BlockSpec((2, _ER, HH), lambda i: (0, i, 0)),
            pl.BlockSpec((2, _ER, HH), lambda i: (0, i, 0)),
        ],
        out_shape=[
            jax.ShapeDtypeStruct((2, NE, HH), F32),
            jax.ShapeDtypeStruct((2, NE, HH), F32),
        ],
    )(eraw, e_in, norm_e, st, bn_g.reshape(1, HID), bn_b.reshape(1, HID),
      cw, cb.reshape(1, HID))


def _head_mlp(s, o, fc1_w, fc1_b, bn1_g, bn1_b, out_w, out_b):
    def body(s_ref, o_ref, w1s_ref, w1o_ref, b1_ref, g_ref, b_ref,
             ow_ref, ob_ref, out_ref):
        f = (_dot(s_ref[...], w1s_ref[...]) + _dot(o_ref[...], w1o_ref[...])
             + b1_ref[...])
        mu = jnp.mean(f, axis=0, keepdims=True)
        var = jnp.mean((f - mu) ** 2, axis=0, keepdims=True)
        f = (f - mu) * lax.rsqrt(var + 1e-5) * g_ref[...] + b_ref[...]
        f = jnp.maximum(f, 0.0)
        out_ref[...] = _dot(f, ow_ref[...]) + ob_ref[...]
    return pl.pallas_call(
        body, out_shape=jax.ShapeDtypeStruct((NTRIP, 1), F32),
    )(s, o, fc1_w[:HID], fc1_w[HID:], fc1_b.reshape(1, -1),
      bn1_g.reshape(1, -1), bn1_b.reshape(1, -1), out_w, out_b.reshape(1, 1))


def kernel(node_id, edge_type, norm_n, norm_e, params, g, triplets):
    p = params
    src = g[0]
    dst = g[1]
    t0 = triplets[:, 0]
    t2 = triplets[:, 2]

    h = _proj_node(node_id, p['lh_w'], p['lh_b'])
    l0 = p['layers'][0]
    e, ce = _proj_edge(edge_type, p['le_w'], p['le_b'], l0['C_w'], l0['C_b'])

    for li in range(len(p['layers'])):
        lp = p['layers'][li]
        ah, dbt, eht = _node_mm(h, lp)
        dbt_f = dbt.reshape(_NC * NN, HID)
        eht_f = eht.reshape(_NC * NN, HH)
        ce_f = ce.reshape(_NC * NE, HH)
        last = li == len(p['layers']) - 1
        if last:
            num, den = _edge_pass_nw(ce_f, dbt_f, eht_f, src, dst)
            eraw = None
        else:
            num, den, eraw = _edge_pass_w(ce_f, dbt_f, eht_f, src, dst)
            eraw = eraw.reshape(2, NE, HH)
        h = _h_update(h, ah, num.reshape(2, NN, HH), den.reshape(2, NN, HH),
                      norm_n, lp['bn_h_g'], lp['bn_h_b'])
        if not last:
            nlp = p['layers'][li + 1]
            st = _e_stats(eraw, norm_e)
            e, ce = _e_apply(eraw, e, norm_e, st, lp['bn_e_g'], lp['bn_e_b'],
                             nlp['C_w'], nlp['C_b'])

    s, o = _head_gather(h, t0, t2)
    out = _head_mlp(s, o, p['fc1_w'], p['fc1_b'], p['bn1_g'], p['bn1_b'],
                    p['out_w'], p['out_b'])
    return (h, out)


# speedup vs baseline: 1.0514x; 1.0514x over previous
import jax
import jax.numpy as jnp
from jax import lax
from jax.experimental import pallas as pl
from jax.experimental.pallas import tpu as pltpu
from jax.experimental.pallas import tpu_sc as plsc

NN = 10000
NE = 320000
HID = 128
HH = 64
NTRIP = 32768
F32 = jnp.float32

_NC, _NS = 2, 16
_CHUNK = 128
_EPT = NE // _NS
_NFULL = _EPT // _CHUNK
_REM = _EPT - _NFULL * _CHUNK
_NNP = 10240
_NPT = _NNP // _NS


def _sc_mesh():
    return plsc.VectorSubcoreMesh(
        core_axis_name="c", subcore_axis_name="s",
        num_cores=_NC, num_subcores=_NS)


def _make_edge_pass(write_eraw: bool):
    outs = [
        jax.ShapeDtypeStruct((_NC * _NNP, HH), F32),
        jax.ShapeDtypeStruct((_NC * _NNP, HH), F32),
    ]
    if write_eraw:
        outs.append(jax.ShapeDtypeStruct((_NC * NE, HH), F32))
    scratch = [
        pltpu.VMEM((_CHUNK,), jnp.int32),
        pltpu.VMEM((_CHUNK,), jnp.int32),
        pltpu.VMEM((_CHUNK,), jnp.int32),
        pltpu.VMEM((_CHUNK, HH), F32),
        pltpu.VMEM((_CHUNK, HID), F32),
        pltpu.VMEM((_CHUNK, HH), F32),
        pltpu.VMEM((_CHUNK, HH), F32),
        pltpu.VMEM_SHARED((_NNP, HH), F32),
        pltpu.VMEM_SHARED((_NNP, HH), F32),
    ]

    def body(ce_hbm, db_hbm, eh_hbm, src_hbm, dst_hbm, *rest):
        if write_eraw:
            (num_hbm, den_hbm, eraw_hbm,
             srcv, dstv, dst2v, cev, dbv, ehv, sigv, num_sp, den_sp) = rest
        else:
            (num_hbm, den_hbm,
             srcv, dstv, dst2v, cev, dbv, ehv, sigv, num_sp, den_sp) = rest
            eraw_hbm = None

        cid = lax.axis_index("c")
        sid = lax.axis_index("s")
        zero16 = jnp.zeros((16,), F32)
        row_off = cid * NN
        ce_off = cid * NE

        def zrow(r, _):
            for v in range(HH // 16):
                sigv[r, pl.ds(v * 16, 16)] = zero16
            return 0
        lax.fori_loop(0, _CHUNK, zrow, 0)
        zbase = sid * _NPT
        for q in range(_NPT // 128):
            pltpu.sync_copy(sigv, num_sp.at[pl.ds(zbase + q * 128, 128)])
            pltpu.sync_copy(sigv, den_sp.at[pl.ds(zbase + q * 128, 128)])
        plsc.subcore_barrier()

        tbase = sid * _EPT

        def chunk(ebase, nrows):
            pltpu.sync_copy(src_hbm.at[pl.ds(ebase, nrows)], srcv.at[pl.ds(0, nrows)])
            pltpu.sync_copy(dst_hbm.at[pl.ds(ebase, nrows)], dstv.at[pl.ds(0, nrows)])
            off_vec = jnp.zeros((16,), jnp.int32) + row_off
            for v in range(_CHUNK // 16):
                sl = pl.ds(v * 16, 16)
                if v * 16 < nrows:
                    srcv[sl] = srcv[sl] + row_off
                    dst2v[sl] = dstv[sl] + row_off
                else:
                    srcv[sl] = off_vec
                    dst2v[sl] = off_vec
            pltpu.sync_copy(ce_hbm.at[pl.ds(ce_off + ebase, nrows)],
                            cev.at[pl.ds(0, nrows)])
            pltpu.sync_copy(db_hbm.at[srcv], dbv)
            pltpu.sync_copy(eh_hbm.at[dst2v], ehv)

            def crow(r, _):
                for v in range(HH // 16):
                    sl = pl.ds(v * 16, 16)
                    e = cev[r, sl] + dbv[r, sl] + ehv[r, sl]
                    cev[r, sl] = e
                    s = 1.0 / (1.0 + jnp.exp(-e))
                    sigv[r, sl] = s
                    ehv[r, sl] = s * dbv[r, pl.ds(HH + v * 16, 16)]
                return 0
            lax.fori_loop(0, nrows, crow, 0)
            if nrows < _CHUNK:
                def zrow2(r, _):
                    for v in range(HH // 16):
                        sl = pl.ds(v * 16, 16)
                        sigv[r, sl] = zero16
                        ehv[r, sl] = zero16
                    return 0
                lax.fori_loop(nrows, _CHUNK, zrow2, 0)
            if eraw_hbm is not None:
                pltpu.sync_copy(cev.at[pl.ds(0, nrows)],
                                eraw_hbm.at[pl.ds(ce_off + ebase, nrows)])
            pltpu.sync_copy(ehv, num_sp.at[dstv], add=True)
            pltpu.sync_copy(sigv, den_sp.at[dstv], add=True)

        def loop_body(c, _):
            chunk(tbase + c * _CHUNK, _CHUNK)
            return 0
        lax.fori_loop(0, _NFULL, loop_body, 0)
        chunk(tbase + _NFULL * _CHUNK, _REM)

        plsc.subcore_barrier()
        fbase = sid * _NPT
        out_off = cid * _NNP
        pltpu.sync_copy(num_sp.at[pl.ds(fbase, _NPT)],
                        num_hbm.at[pl.ds(out_off + fbase, _NPT)])
        pltpu.sync_copy(den_sp.at[pl.ds(fbase, _NPT)],
                        den_hbm.at[pl.ds(out_off + fbase, _NPT)])

    return pl.kernel(body, out_type=tuple(outs), mesh=_sc_mesh(),
                     scratch_types=scratch,
                     compiler_params=pltpu.CompilerParams(
                         use_tc_tiling_on_sc=False))


_edge_pass_w = _make_edge_pass(True)
_edge_pass_nw = _make_edge_pass(False)


_TPW = NTRIP // (_NC * _NS)
_TCH = _TPW // _CHUNK


def _make_head_gather():
    outs = (jax.ShapeDtypeStruct((NTRIP, HID), F32),
            jax.ShapeDtypeStruct((NTRIP, HID), F32))
    scratch = [
        pltpu.VMEM((_CHUNK,), jnp.int32),
        pltpu.VMEM((_CHUNK, HID), F32),
    ]

    def body(h_hbm, t0_hbm, t2_hbm, s_hbm, o_hbm, idxv, rowsv):
        cid = lax.axis_index("c")
        sid = lax.axis_index("s")
        base = (sid * _NC + cid) * _TPW
        for t_hbm, out_hbm in ((t0_hbm, s_hbm), (t2_hbm, o_hbm)):
            def cb(c, _, t_hbm=t_hbm, out_hbm=out_hbm):
                rb = base + c * _CHUNK
                pltpu.sync_copy(t_hbm.at[pl.ds(rb, _CHUNK)], idxv)
                pltpu.sync_copy(h_hbm.at[idxv], rowsv)
                pltpu.sync_copy(rowsv, out_hbm.at[pl.ds(rb, _CHUNK)])
                return 0
            lax.fori_loop(0, _TCH, cb, 0)

    return pl.kernel(body, out_type=outs, mesh=_sc_mesh(),
                     scratch_types=scratch)


_head_gather = _make_head_gather()


def _dot(a, b):
    return jnp.dot(a, b, preferred_element_type=F32)


def _proj_node(x, w, b):
    def body(x_ref, w_ref, b_ref, o_ref):
        o_ref[...] = _dot(x_ref[...], w_ref[...]) + b_ref[...]
    return pl.pallas_call(
        body, out_shape=jax.ShapeDtypeStruct((NN, HID), F32),
    )(x, w, b.reshape(1, HID))


_ER = 3200
_EG = NE // _ER


def _proj_edge(et, lw, lb, cw, cb):
    def body(x_ref, lw_ref, lb_ref, cw_ref, cb_ref, e_ref, ce_ref):
        e = _dot(x_ref[...], lw_ref[...]) + lb_ref[...]
        e_ref[0] = e[:, :HH]
        e_ref[1] = e[:, HH:]
        ce = _dot(e, cw_ref[...]) + cb_ref[...]
        ce_ref[0] = ce[:, :HH]
        ce_ref[1] = ce[:, HH:]
    return pl.pallas_call(
        body,
        grid=(_EG,),
        in_specs=[
            pl.BlockSpec((_ER, 16), lambda i: (i, 0)),
            pl.BlockSpec((16, HID), lambda i: (0, 0)),
            pl.BlockSpec((1, HID), lambda i: (0, 0)),
            pl.BlockSpec((HID, HID), lambda i: (0, 0)),
            pl.BlockSpec((1, HID), lambda i: (0, 0)),
        ],
        out_specs=[
            pl.BlockSpec((2, _ER, HH), lambda i: (0, i, 0)),
            pl.BlockSpec((2, _ER, HH), lambda i: (0, i, 0)),
        ],
        out_shape=[
            jax.ShapeDtypeStruct((2, NE, HH), F32),
            jax.ShapeDtypeStruct((2, NE, HH), F32),
        ],
    )(et, lw, lb.reshape(1, HID), cw, cb.reshape(1, HID))


def _node_mm(h, lp):
    def body(h_ref, aw, ab, bw, bb, dw, dbias, ew, eb, ah_ref, dbt_ref, eht_ref):
        hh = h_ref[...]
        Ah = _dot(hh, aw[...]) + ab[...]
        Bh = _dot(hh, bw[...]) + bb[...]
        Dh = _dot(hh, dw[...]) + dbias[...]
        Eh = _dot(hh, ew[...]) + eb[...]
        ah_ref[...] = Ah
        dbt_ref[0] = jnp.concatenate([Dh[:, :HH], Bh[:, :HH]], axis=1)
        dbt_ref[1] = jnp.concatenate([Dh[:, HH:], Bh[:, HH:]], axis=1)
        eht_ref[0] = Eh[:, :HH]
        eht_ref[1] = Eh[:, HH:]
    return pl.pallas_call(
        body,
        out_shape=[
            jax.ShapeDtypeStruct((NN, HID), F32),
            jax.ShapeDtypeStruct((2, NN, HID), F32),
            jax.ShapeDtypeStruct((2, NN, HH), F32),
        ],
    )(h, lp['A_w'], lp['A_b'].reshape(1, HID),
      lp['B_w'], lp['B_b'].reshape(1, HID),
      lp['D_w'], lp['D_b'].reshape(1, HID),
      lp['E_w'], lp['E_b'].reshape(1, HID))


def _h_update(h_in, ah, num, den, norm_n, bn_g, bn_b):
    def body(hin_ref, ah_ref, num_ref, den_ref, nn_ref, g_ref, b_ref, out_ref):
        num_f = jnp.concatenate([num_ref[0, :NN], num_ref[1, :NN]], axis=1)
        den_f = jnp.concatenate([den_ref[0, :NN], den_ref[1, :NN]], axis=1)
        hn = ah_ref[...] + num_f / (den_f + 1e-6)
        hn = hn * nn_ref[...]
        mu = jnp.mean(hn, axis=0, keepdims=True)
        var = jnp.mean((hn - mu) ** 2, axis=0, keepdims=True)
        hn = (hn - mu) * lax.rsqrt(var + 1e-5) * g_ref[...] + b_ref[...]
        out_ref[...] = hin_ref[...] + jnp.maximum(hn, 0.0)
    return pl.pallas_call(
        body, out_shape=jax.ShapeDtypeStruct((NN, HID), F32),
    )(h_in, ah, num, den, norm_n, bn_g.reshape(1, HID), bn_b.reshape(1, HID))


def _e_stats(eraw, norm_e):
    def body(er_ref, ne_ref, out_ref):
        i = pl.program_id(0)
        y = jnp.concatenate([er_ref[0], er_ref[1]], axis=1) * ne_ref[...]
        @pl.when(i == 0)
        def _():
            out_ref[...] = jnp.zeros_like(out_ref)
        out_ref[0:1, :] += jnp.sum(y, axis=0, keepdims=True)
        out_ref[1:2, :] += jnp.sum(y * y, axis=0, keepdims=True)
    return pl.pallas_call(
        body,
        grid=(_EG,),
        in_specs=[
            pl.BlockSpec((2, _ER, HH), lambda i: (0, i, 0)),
            pl.BlockSpec((_ER, 1), lambda i: (i, 0)),
        ],
        out_specs=pl.BlockSpec((8, HID), lambda i: (0, 0)),
        out_shape=jax.ShapeDtypeStruct((8, HID), F32),
    )(eraw, norm_e)


def _e_apply(eraw, e_in, norm_e, st, bn_g, bn_b, cw, cb):
    def body(er_ref, ein_ref, ne_ref, st_ref, g_ref, b_ref, cw_ref, cb_ref,
             enew_ref, ce_ref):
        y = jnp.concatenate([er_ref[0], er_ref[1]], axis=1) * ne_ref[...]
        mu = st_ref[0:1, :] * (1.0 / NE)
        var = st_ref[1:2, :] * (1.0 / NE) - mu * mu
        z = (y - mu) * lax.rsqrt(var + 1e-5) * g_ref[...] + b_ref[...]
        z = jnp.maximum(z, 0.0)
        enew = jnp.concatenate([ein_ref[0], ein_ref[1]], axis=1) + z
        enew_ref[0] = enew[:, :HH]
        enew_ref[1] = enew[:, HH:]
        ce = _dot(enew, cw_ref[...]) + cb_ref[...]
        ce_ref[0] = ce[:, :HH]
        ce_ref[1] = ce[:, HH:]
    return pl.pallas_call(
        body,
        grid=(_EG,),
        in_specs=[
            pl.BlockSpec((2, _ER, HH), lambda i: (0, i, 0)),
            pl.BlockSpec((2, _ER, HH), lambda i: (0, i, 0)),
            pl.BlockSpec((_ER, 1), lambda i: (i, 0)),
            pl.BlockSpec((8, HID), lambda i: (0, 0)),
            pl.BlockSpec((1, HID), lambda i: (0, 0)),
            pl.BlockSpec((1, HID), lambda i: (0, 0)),
            pl.BlockSpec((HID, HID), lambda i: (0, 0)),
            pl.BlockSpec((1, HID), lambda i: (0, 0)),
        ],
        out_specs=[
            pl.BlockSpec((2, _ER, HH), lambda i: (0, i, 0)),
            pl.BlockSpec((2, _ER, HH), lambda i: (0, i, 0)),
        ],
        out_shape=[
            jax.ShapeDtypeStruct((2, NE, HH), F32),
            jax.ShapeDtypeStruct((2, NE, HH), F32),
        ],
    )(eraw, e_in, norm_e, st, bn_g.reshape(1, HID), bn_b.reshape(1, HID),
      cw, cb.reshape(1, HID))


_HR = 4096
_HG = NTRIP // _HR


def _head_mlp(s, o, fc1_w, fc1_b, bn1_g, bn1_b, out_w, out_b):
    def body1(s_ref, o_ref, w1s_ref, w1o_ref, b1_ref, f_ref, st_ref):
        i = pl.program_id(0)
        f = (_dot(s_ref[...], w1s_ref[...]) + _dot(o_ref[...], w1o_ref[...])
             + b1_ref[...])
        f_ref[...] = f
        @pl.when(i == 0)
        def _():
            st_ref[...] = jnp.zeros_like(st_ref)
        st_ref[0:1, :] += jnp.sum(f, axis=0, keepdims=True)
        st_ref[1:2, :] += jnp.sum(f * f, axis=0, keepdims=True)

    f, st = pl.pallas_call(
        body1,
        grid=(_HG,),
        in_specs=[
            pl.BlockSpec((_HR, HID), lambda i: (i, 0)),
            pl.BlockSpec((_HR, HID), lambda i: (i, 0)),
            pl.BlockSpec((HID, 200), lambda i: (0, 0)),
            pl.BlockSpec((HID, 200), lambda i: (0, 0)),
            pl.BlockSpec((1, 200), lambda i: (0, 0)),
        ],
        out_specs=[
            pl.BlockSpec((_HR, 200), lambda i: (i, 0)),
            pl.BlockSpec((8, 200), lambda i: (0, 0)),
        ],
        out_shape=[
            jax.ShapeDtypeStruct((NTRIP, 200), F32),
            jax.ShapeDtypeStruct((8, 200), F32),
        ],
    )(s, o, fc1_w[:HID], fc1_w[HID:], fc1_b.reshape(1, -1))

    def body2(f_ref, st_ref, g_ref, b_ref, ow_ref, ob_ref, out_ref):
        mu = st_ref[0:1, :] * (1.0 / NTRIP)
        var = st_ref[1:2, :] * (1.0 / NTRIP) - mu * mu
        z = (f_ref[...] - mu) * lax.rsqrt(var + 1e-5) * g_ref[...] + b_ref[...]
        z = jnp.maximum(z, 0.0)
        out_ref[...] = _dot(z, ow_ref[...]) + ob_ref[...]

    return pl.pallas_call(
        body2,
        grid=(_HG,),
        in_specs=[
            pl.BlockSpec((_HR, 200), lambda i: (i, 0)),
            pl.BlockSpec((8, 200), lambda i: (0, 0)),
            pl.BlockSpec((1, 200), lambda i: (0, 0)),
            pl.BlockSpec((1, 200), lambda i: (0, 0)),
            pl.BlockSpec((200, 1), lambda i: (0, 0)),
            pl.BlockSpec((1, 1), lambda i: (0, 0)),
        ],
        out_specs=pl.BlockSpec((_HR, 1), lambda i: (i, 0)),
        out_shape=jax.ShapeDtypeStruct((NTRIP, 1), F32),
    )(f, st, bn1_g.reshape(1, -1), bn1_b.reshape(1, -1), out_w,
      out_b.reshape(1, 1))


def kernel(node_id, edge_type, norm_n, norm_e, params, g, triplets):
    p = params
    src = g[0]
    dst = g[1]
    t0 = triplets[:, 0]
    t2 = triplets[:, 2]

    h = _proj_node(node_id, p['lh_w'], p['lh_b'])
    l0 = p['layers'][0]
    e, ce = _proj_edge(edge_type, p['le_w'], p['le_b'], l0['C_w'], l0['C_b'])

    for li in range(len(p['layers'])):
        lp = p['layers'][li]
        ah, dbt, eht = _node_mm(h, lp)
        dbt_f = dbt.reshape(_NC * NN, HID)
        eht_f = eht.reshape(_NC * NN, HH)
        ce_f = ce.reshape(_NC * NE, HH)
        last = li == len(p['layers']) - 1
        if last:
            num, den = _edge_pass_nw(ce_f, dbt_f, eht_f, src, dst)
            eraw = None
        else:
            num, den, eraw = _edge_pass_w(ce_f, dbt_f, eht_f, src, dst)
            eraw = eraw.reshape(2, NE, HH)
        h = _h_update(h, ah, num.reshape(2, _NNP, HH), den.reshape(2, _NNP, HH),
                      norm_n, lp['bn_h_g'], lp['bn_h_b'])
        if not last:
            nlp = p['layers'][li + 1]
            st = _e_stats(eraw, norm_e)
            e, ce = _e_apply(eraw, e, norm_e, st, lp['bn_e_g'], lp['bn_e_b'],
                             nlp['C_w'], nlp['C_b'])

    s, o = _head_gather(h, t0, t2)
    out = _head_mlp(s, o, p['fc1_w'], p['fc1_b'], p['bn1_g'], p['bn1_b'],
                    p['out_w'], p['out_b'])
    return (h, out)


# speedup vs baseline: 1.9440x; 1.8491x over previous
import jax
import jax.numpy as jnp
from jax import lax
from jax.experimental import pallas as pl
from jax.experimental.pallas import tpu as pltpu
from jax.experimental.pallas import tpu_sc as plsc

NN = 10000
NE = 320000
HID = 128
HH = 64
NTRIP = 32768
F32 = jnp.float32

_NC, _NS = 2, 16
_CHUNK = 128
_EPT = NE // _NS
_NFULL = _EPT // _CHUNK
_REM = _EPT - _NFULL * _CHUNK
_NNP = 10240
_NPT = _NNP // _NS


def _sc_mesh():
    return plsc.VectorSubcoreMesh(
        core_axis_name="c", subcore_axis_name="s",
        num_cores=_NC, num_subcores=_NS)


def _make_edge_pass(write_eraw: bool):
    outs = [
        jax.ShapeDtypeStruct((_NC * _NNP, HH), F32),
        jax.ShapeDtypeStruct((_NC * _NNP, HH), F32),
    ]
    if write_eraw:
        outs.append(jax.ShapeDtypeStruct((_NC * NE, HH), F32))
    scratch = [
        pltpu.VMEM((_CHUNK,), jnp.int32),
        pltpu.VMEM((_CHUNK,), jnp.int32),
        pltpu.VMEM((_CHUNK,), jnp.int32),
        pltpu.VMEM((_CHUNK, HH), F32),
        pltpu.VMEM((_CHUNK, HID), F32),
        pltpu.VMEM((_CHUNK, HH), F32),
        pltpu.VMEM((_CHUNK, HH), F32),
        pltpu.VMEM_SHARED((_NNP, HH), F32),
        pltpu.VMEM_SHARED((_NNP, HH), F32),
    ]

    def body(ce_hbm, db_hbm, eh_hbm, src_hbm, dst_hbm, *rest):
        if write_eraw:
            (num_hbm, den_hbm, eraw_hbm,
             srcv, dstv, dst2v, cev, dbv, ehv, sigv, num_sp, den_sp) = rest
        else:
            (num_hbm, den_hbm,
             srcv, dstv, dst2v, cev, dbv, ehv, sigv, num_sp, den_sp) = rest
            eraw_hbm = None

        cid = lax.axis_index("c")
        sid = lax.axis_index("s")
        zero16 = jnp.zeros((16,), F32)
        row_off = cid * NN
        ce_off = cid * NE

        def zrow(r, _):
            for v in range(HH // 16):
                sigv[r, pl.ds(v * 16, 16)] = zero16
            return 0
        lax.fori_loop(0, _CHUNK, zrow, 0)
        zbase = sid * _NPT
        for q in range(_NPT // 128):
            pltpu.sync_copy(sigv, num_sp.at[pl.ds(zbase + q * 128, 128)])
            pltpu.sync_copy(sigv, den_sp.at[pl.ds(zbase + q * 128, 128)])
        plsc.subcore_barrier()

        tbase = sid * _EPT

        def chunk(ebase, nrows):
            pltpu.sync_copy(src_hbm.at[pl.ds(ebase, nrows)], srcv.at[pl.ds(0, nrows)])
            pltpu.sync_copy(dst_hbm.at[pl.ds(ebase, nrows)], dstv.at[pl.ds(0, nrows)])
            off_vec = jnp.zeros((16,), jnp.int32) + row_off
            for v in range(_CHUNK // 16):
                sl = pl.ds(v * 16, 16)
                if v * 16 < nrows:
                    srcv[sl] = srcv[sl] + row_off
                    dst2v[sl] = dstv[sl] + row_off
                else:
                    srcv[sl] = off_vec
                    dst2v[sl] = off_vec
            pltpu.sync_copy(ce_hbm.at[pl.ds(ce_off + ebase, nrows)],
                            cev.at[pl.ds(0, nrows)])
            pltpu.sync_copy(db_hbm.at[srcv], dbv)
            pltpu.sync_copy(eh_hbm.at[dst2v], ehv)

            @plsc.parallel_loop(0, nrows, unroll=4)
            def _crow(r):
                for v in range(HH // 16):
                    sl = pl.ds(v * 16, 16)
                    e = cev[r, sl] + dbv[r, sl] + ehv[r, sl]
                    cev[r, sl] = e
                    s = 1.0 / (1.0 + jnp.exp(-e))
                    sigv[r, sl] = s
                    ehv[r, sl] = s * dbv[r, pl.ds(HH + v * 16, 16)]
            if nrows < _CHUNK:
                @plsc.parallel_loop(nrows, _CHUNK, unroll=4)
                def _zrow2(r):
                    for v in range(HH // 16):
                        sl = pl.ds(v * 16, 16)
                        sigv[r, sl] = zero16
                        ehv[r, sl] = zero16
            if eraw_hbm is not None:
                pltpu.sync_copy(cev.at[pl.ds(0, nrows)],
                                eraw_hbm.at[pl.ds(ce_off + ebase, nrows)])
            pltpu.sync_copy(ehv, num_sp.at[dstv], add=True)
            pltpu.sync_copy(sigv, den_sp.at[dstv], add=True)

        def loop_body(c, _):
            chunk(tbase + c * _CHUNK, _CHUNK)
            return 0
        lax.fori_loop(0, _NFULL, loop_body, 0)
        chunk(tbase + _NFULL * _CHUNK, _REM)

        plsc.subcore_barrier()
        fbase = sid * _NPT
        out_off = cid * _NNP
        pltpu.sync_copy(num_sp.at[pl.ds(fbase, _NPT)],
                        num_hbm.at[pl.ds(out_off + fbase, _NPT)])
        pltpu.sync_copy(den_sp.at[pl.ds(fbase, _NPT)],
                        den_hbm.at[pl.ds(out_off + fbase, _NPT)])

    return pl.kernel(body, out_type=tuple(outs), mesh=_sc_mesh(),
                     scratch_types=scratch,
                     compiler_params=pltpu.CompilerParams(
                         use_tc_tiling_on_sc=False))


_edge_pass_w = _make_edge_pass(True)
_edge_pass_nw = _make_edge_pass(False)


_TPW = NTRIP // (_NC * _NS)
_TCH = _TPW // _CHUNK


def _make_head_gather():
    outs = (jax.ShapeDtypeStruct((NTRIP, HID), F32),
            jax.ShapeDtypeStruct((NTRIP, HID), F32))
    scratch = [
        pltpu.VMEM((_CHUNK,), jnp.int32),
        pltpu.VMEM((_CHUNK, HID), F32),
    ]

    def body(h_hbm, t0_hbm, t2_hbm, s_hbm, o_hbm, idxv, rowsv):
        cid = lax.axis_index("c")
        sid = lax.axis_index("s")
        base = (sid * _NC + cid) * _TPW
        for t_hbm, out_hbm in ((t0_hbm, s_hbm), (t2_hbm, o_hbm)):
            def cb(c, _, t_hbm=t_hbm, out_hbm=out_hbm):
                rb = base + c * _CHUNK
                pltpu.sync_copy(t_hbm.at[pl.ds(rb, _CHUNK)], idxv)
                pltpu.sync_copy(h_hbm.at[idxv], rowsv)
                pltpu.sync_copy(rowsv, out_hbm.at[pl.ds(rb, _CHUNK)])
                return 0
            lax.fori_loop(0, _TCH, cb, 0)

    return pl.kernel(body, out_type=outs, mesh=_sc_mesh(),
                     scratch_types=scratch)


_head_gather = _make_head_gather()


def _dot(a, b):
    return jnp.dot(a, b, preferred_element_type=F32)


def _proj_node(x, w, b):
    def body(x_ref, w_ref, b_ref, o_ref):
        o_ref[...] = _dot(x_ref[...], w_ref[...]) + b_ref[...]
    return pl.pallas_call(
        body, out_shape=jax.ShapeDtypeStruct((NN, HID), F32),
    )(x, w, b.reshape(1, HID))


_ER = 3200
_EG = NE // _ER


def _proj_edge(et, lw, lb, cw, cb):
    def body(x_ref, lw_ref, lb_ref, cw_ref, cb_ref, e_ref, ce_ref):
        e = _dot(x_ref[...], lw_ref[...]) + lb_ref[...]
        e_ref[0] = e[:, :HH]
        e_ref[1] = e[:, HH:]
        ce = _dot(e, cw_ref[...]) + cb_ref[...]
        ce_ref[0] = ce[:, :HH]
        ce_ref[1] = ce[:, HH:]
    return pl.pallas_call(
        body,
        grid=(_EG,),
        in_specs=[
            pl.BlockSpec((_ER, 16), lambda i: (i, 0)),
            pl.BlockSpec((16, HID), lambda i: (0, 0)),
            pl.BlockSpec((1, HID), lambda i: (0, 0)),
            pl.BlockSpec((HID, HID), lambda i: (0, 0)),
            pl.BlockSpec((1, HID), lambda i: (0, 0)),
        ],
        out_specs=[
            pl.BlockSpec((2, _ER, HH), lambda i: (0, i, 0)),
            pl.BlockSpec((2, _ER, HH), lambda i: (0, i, 0)),
        ],
        out_shape=[
            jax.ShapeDtypeStruct((2, NE, HH), F32),
            jax.ShapeDtypeStruct((2, NE, HH), F32),
        ],
    )(et, lw, lb.reshape(1, HID), cw, cb.reshape(1, HID))


def _node_mm(h, lp):
    def body(h_ref, aw, ab, bw, bb, dw, dbias, ew, eb, ah_ref, dbt_ref, eht_ref):
        hh = h_ref[...]
        Ah = _dot(hh, aw[...]) + ab[...]
        Bh = _dot(hh, bw[...]) + bb[...]
        Dh = _dot(hh, dw[...]) + dbias[...]
        Eh = _dot(hh, ew[...]) + eb[...]
        ah_ref[...] = Ah
        dbt_ref[0] = jnp.concatenate([Dh[:, :HH], Bh[:, :HH]], axis=1)
        dbt_ref[1] = jnp.concatenate([Dh[:, HH:], Bh[:, HH:]], axis=1)
        eht_ref[0] = Eh[:, :HH]
        eht_ref[1] = Eh[:, HH:]
    return pl.pallas_call(
        body,
        out_shape=[
            jax.ShapeDtypeStruct((NN, HID), F32),
            jax.ShapeDtypeStruct((2, NN, HID), F32),
            jax.ShapeDtypeStruct((2, NN, HH), F32),
        ],
    )(h, lp['A_w'], lp['A_b'].reshape(1, HID),
      lp['B_w'], lp['B_b'].reshape(1, HID),
      lp['D_w'], lp['D_b'].reshape(1, HID),
      lp['E_w'], lp['E_b'].reshape(1, HID))


def _h_update(h_in, ah, num, den, norm_n, bn_g, bn_b):
    def body(hin_ref, ah_ref, num_ref, den_ref, nn_ref, g_ref, b_ref, out_ref):
        num_f = jnp.concatenate([num_ref[0, :NN], num_ref[1, :NN]], axis=1)
        den_f = jnp.concatenate([den_ref[0, :NN], den_ref[1, :NN]], axis=1)
        hn = ah_ref[...] + num_f / (den_f + 1e-6)
        hn = hn * nn_ref[...]
        mu = jnp.mean(hn, axis=0, keepdims=True)
        var = jnp.mean((hn - mu) ** 2, axis=0, keepdims=True)
        hn = (hn - mu) * lax.rsqrt(var + 1e-5) * g_ref[...] + b_ref[...]
        out_ref[...] = hin_ref[...] + jnp.maximum(hn, 0.0)
    return pl.pallas_call(
        body, out_shape=jax.ShapeDtypeStruct((NN, HID), F32),
    )(h_in, ah, num, den, norm_n, bn_g.reshape(1, HID), bn_b.reshape(1, HID))


def _e_stats(eraw, norm_e):
    def body(er_ref, ne_ref, out_ref):
        i = pl.program_id(0)
        y = jnp.concatenate([er_ref[0], er_ref[1]], axis=1) * ne_ref[...]
        @pl.when(i == 0)
        def _():
            out_ref[...] = jnp.zeros_like(out_ref)
        out_ref[0:1, :] += jnp.sum(y, axis=0, keepdims=True)
        out_ref[1:2, :] += jnp.sum(y * y, axis=0, keepdims=True)
    return pl.pallas_call(
        body,
        grid=(_EG,),
        in_specs=[
            pl.BlockSpec((2, _ER, HH), lambda i: (0, i, 0)),
            pl.BlockSpec((_ER, 1), lambda i: (i, 0)),
        ],
        out_specs=pl.BlockSpec((8, HID), lambda i: (0, 0)),
        out_shape=jax.ShapeDtypeStruct((8, HID), F32),
    )(eraw, norm_e)


def _e_apply(eraw, e_in, norm_e, st, bn_g, bn_b, cw, cb):
    def body(er_ref, ein_ref, ne_ref, st_ref, g_ref, b_ref, cw_ref, cb_ref,
             enew_ref, ce_ref):
        y = jnp.concatenate([er_ref[0], er_ref[1]], axis=1) * ne_ref[...]
        mu = st_ref[0:1, :] * (1.0 / NE)
        var = st_ref[1:2, :] * (1.0 / NE) - mu * mu
        z = (y - mu) * lax.rsqrt(var + 1e-5) * g_ref[...] + b_ref[...]
        z = jnp.maximum(z, 0.0)
        enew = jnp.concatenate([ein_ref[0], ein_ref[1]], axis=1) + z
        enew_ref[0] = enew[:, :HH]
        enew_ref[1] = enew[:, HH:]
        ce = _dot(enew, cw_ref[...]) + cb_ref[...]
        ce_ref[0] = ce[:, :HH]
        ce_ref[1] = ce[:, HH:]
    return pl.pallas_call(
        body,
        grid=(_EG,),
        in_specs=[
            pl.BlockSpec((2, _ER, HH), lambda i: (0, i, 0)),
            pl.BlockSpec((2, _ER, HH), lambda i: (0, i, 0)),
            pl.BlockSpec((_ER, 1), lambda i: (i, 0)),
            pl.BlockSpec((8, HID), lambda i: (0, 0)),
            pl.BlockSpec((1, HID), lambda i: (0, 0)),
            pl.BlockSpec((1, HID), lambda i: (0, 0)),
            pl.BlockSpec((HID, HID), lambda i: (0, 0)),
            pl.BlockSpec((1, HID), lambda i: (0, 0)),
        ],
        out_specs=[
            pl.BlockSpec((2, _ER, HH), lambda i: (0, i, 0)),
            pl.BlockSpec((2, _ER, HH), lambda i: (0, i, 0)),
        ],
        out_shape=[
            jax.ShapeDtypeStruct((2, NE, HH), F32),
            jax.ShapeDtypeStruct((2, NE, HH), F32),
        ],
    )(eraw, e_in, norm_e, st, bn_g.reshape(1, HID), bn_b.reshape(1, HID),
      cw, cb.reshape(1, HID))


_HR = 4096
_HG = NTRIP // _HR


def _head_mlp(s, o, fc1_w, fc1_b, bn1_g, bn1_b, out_w, out_b):
    def body1(s_ref, o_ref, w1s_ref, w1o_ref, b1_ref, f_ref, st_ref):
        i = pl.program_id(0)
        f = (_dot(s_ref[...], w1s_ref[...]) + _dot(o_ref[...], w1o_ref[...])
             + b1_ref[...])
        f_ref[...] = f
        @pl.when(i == 0)
        def _():
            st_ref[...] = jnp.zeros_like(st_ref)
        st_ref[0:1, :] += jnp.sum(f, axis=0, keepdims=True)
        st_ref[1:2, :] += jnp.sum(f * f, axis=0, keepdims=True)

    f, st = pl.pallas_call(
        body1,
        grid=(_HG,),
        in_specs=[
            pl.BlockSpec((_HR, HID), lambda i: (i, 0)),
            pl.BlockSpec((_HR, HID), lambda i: (i, 0)),
            pl.BlockSpec((HID, 200), lambda i: (0, 0)),
            pl.BlockSpec((HID, 200), lambda i: (0, 0)),
            pl.BlockSpec((1, 200), lambda i: (0, 0)),
        ],
        out_specs=[
            pl.BlockSpec((_HR, 200), lambda i: (i, 0)),
            pl.BlockSpec((8, 200), lambda i: (0, 0)),
        ],
        out_shape=[
            jax.ShapeDtypeStruct((NTRIP, 200), F32),
            jax.ShapeDtypeStruct((8, 200), F32),
        ],
    )(s, o, fc1_w[:HID], fc1_w[HID:], fc1_b.reshape(1, -1))

    def body2(f_ref, st_ref, g_ref, b_ref, ow_ref, ob_ref, out_ref):
        mu = st_ref[0:1, :] * (1.0 / NTRIP)
        var = st_ref[1:2, :] * (1.0 / NTRIP) - mu * mu
        z = (f_ref[...] - mu) * lax.rsqrt(var + 1e-5) * g_ref[...] + b_ref[...]
        z = jnp.maximum(z, 0.0)
        out_ref[...] = _dot(z, ow_ref[...]) + ob_ref[...]

    return pl.pallas_call(
        body2,
        grid=(_HG,),
        in_specs=[
            pl.BlockSpec((_HR, 200), lambda i: (i, 0)),
            pl.BlockSpec((8, 200), lambda i: (0, 0)),
            pl.BlockSpec((1, 200), lambda i: (0, 0)),
            pl.BlockSpec((1, 200), lambda i: (0, 0)),
            pl.BlockSpec((200, 1), lambda i: (0, 0)),
            pl.BlockSpec((1, 1), lambda i: (0, 0)),
        ],
        out_specs=pl.BlockSpec((_HR, 1), lambda i: (i, 0)),
        out_shape=jax.ShapeDtypeStruct((NTRIP, 1), F32),
    )(f, st, bn1_g.reshape(1, -1), bn1_b.reshape(1, -1), out_w,
      out_b.reshape(1, 1))


def kernel(node_id, edge_type, norm_n, norm_e, params, g, triplets):
    p = params
    src = g[0]
    dst = g[1]
    t0 = triplets[:, 0]
    t2 = triplets[:, 2]

    h = _proj_node(node_id, p['lh_w'], p['lh_b'])
    l0 = p['layers'][0]
    e, ce = _proj_edge(edge_type, p['le_w'], p['le_b'], l0['C_w'], l0['C_b'])

    for li in range(len(p['layers'])):
        lp = p['layers'][li]
        ah, dbt, eht = _node_mm(h, lp)
        dbt_f = dbt.reshape(_NC * NN, HID)
        eht_f = eht.reshape(_NC * NN, HH)
        ce_f = ce.reshape(_NC * NE, HH)
        last = li == len(p['layers']) - 1
        if last:
            num, den = _edge_pass_nw(ce_f, dbt_f, eht_f, src, dst)
            eraw = None
        else:
            num, den, eraw = _edge_pass_w(ce_f, dbt_f, eht_f, src, dst)
            eraw = eraw.reshape(2, NE, HH)
        h = _h_update(h, ah, num.reshape(2, _NNP, HH), den.reshape(2, _NNP, HH),
                      norm_n, lp['bn_h_g'], lp['bn_h_b'])
        if not last:
            nlp = p['layers'][li + 1]
            st = _e_stats(eraw, norm_e)
            e, ce = _e_apply(eraw, e, norm_e, st, lp['bn_e_g'], lp['bn_e_b'],
                             nlp['C_w'], nlp['C_b'])

    s, o = _head_gather(h, t0, t2)
    out = _head_mlp(s, o, p['fc1_w'], p['fc1_b'], p['bn1_g'], p['bn1_b'],
                    p['out_w'], p['out_b'])
    return (h, out)


# speedup vs baseline: 2.6546x; 1.3655x over previous
import jax
import jax.numpy as jnp
from jax import lax
from jax.experimental import pallas as pl
from jax.experimental.pallas import tpu as pltpu
from jax.experimental.pallas import tpu_sc as plsc

NN = 10000
NE = 320000
HID = 128
HH = 64
NTRIP = 32768
F32 = jnp.float32

_NC, _NS = 2, 16
_CHUNK = 128
_EPT = NE // _NS
_NFULL = _EPT // _CHUNK
_REM = _EPT - _NFULL * _CHUNK
_NNP = 10240
_NPT = _NNP // _NS


def _sc_mesh():
    return plsc.VectorSubcoreMesh(
        core_axis_name="c", subcore_axis_name="s",
        num_cores=_NC, num_subcores=_NS)


_EC = 48
_ENF = _EPT // _EC
_EREM = _EPT - _ENF * _EC
_NCT = _ENF + 1
_NSLOT = 3
_ISLOT = 6


def _make_edge_pass(write_eraw: bool):
    outs = [
        jax.ShapeDtypeStruct((_NC * _NNP, HID), F32),
    ]
    if write_eraw:
        outs.append(jax.ShapeDtypeStruct((_NC * NE, HH), F32))
    scratch = [
        pltpu.VMEM((_ISLOT, 2, _EC), jnp.int32),
        pltpu.VMEM((_ISLOT, _EC), jnp.int32),
        pltpu.VMEM((_NSLOT, _EC, HH), F32),
        pltpu.VMEM((_NSLOT, _EC, HID), F32),
        pltpu.VMEM((_NSLOT, _EC, HH), F32),
        pltpu.VMEM_SHARED((_NNP, HID), F32),
    ] + [pltpu.SemaphoreType.DMA] * (2 * _NSLOT + _ISLOT)

    def body(ce_hbm, db_hbm, eh_hbm, g_hbm, *rest):
        if write_eraw:
            (acc_hbm, eraw_hbm, idxv, dst2v, cev, dbv, ehv, acc_sp,
             *sems) = rest
        else:
            (acc_hbm, idxv, dst2v, cev, dbv, ehv, acc_sp, *sems) = rest
            eraw_hbm = None
        gsem = sems[:_NSLOT]
        wsem = sems[_NSLOT:2 * _NSLOT]
        isem = sems[2 * _NSLOT:]

        cid = lax.axis_index("c")
        sid = lax.axis_index("s")
        zero16 = jnp.zeros((16,), F32)
        row_off = cid * NN
        ce_off = cid * NE

        @plsc.parallel_loop(0, _EC, unroll=4)
        def _zrow(r):
            for v in range(HID // 16):
                dbv[0, r, pl.ds(v * 16, 16)] = zero16
        zbase = sid * _NPT
        for nr, qo in [(_EC, 48 * q) for q in range(13)] + [(16, 624)]:
            pltpu.sync_copy(dbv.at[0, pl.ds(0, nr)],
                            acc_sp.at[pl.ds(zbase + qo, nr)])
        plsc.subcore_barrier()

        tbase = sid * _EPT

        def nrows_of(c):
            return _EREM if (isinstance(c, int) and c == _NCT - 1) else _EC

        def issue_idx(c, q):
            n = nrows_of(c)
            ebase = tbase + c * _EC
            pltpu.async_copy(g_hbm.at[:, pl.ds(ebase, n)],
                             idxv.at[q, :, pl.ds(0, n)], isem[q])

        def prep_gathers(c, q, s):
            n = nrows_of(c)
            ebase = tbase + c * _EC
            pltpu.make_async_copy(g_hbm.at[:, pl.ds(ebase, n)],
                                  idxv.at[q, :, pl.ds(0, n)], isem[q]).wait()
            off_vec = jnp.zeros((16,), jnp.int32) + row_off
            for v in range(_EC // 16):
                sl = pl.ds(v * 16, 16)
                if v * 16 < n:
                    idxv[q, 0, sl] = idxv[q, 0, sl] + row_off
                    dst2v[q, sl] = idxv[q, 1, sl] + row_off
                else:
                    idxv[q, 0, sl] = off_vec
                    dst2v[q, sl] = off_vec
            pltpu.async_copy(ce_hbm.at[pl.ds(ce_off + ebase, n)],
                             cev.at[s, pl.ds(0, n)], gsem[s])
            pltpu.async_copy(db_hbm.at[idxv.at[q, 0]], dbv.at[s], gsem[s])
            pltpu.async_copy(eh_hbm.at[dst2v.at[q]], ehv.at[s], gsem[s])

        def wait_gathers(c, q, s):
            n = nrows_of(c)
            ebase = tbase + c * _EC
            pltpu.make_async_copy(ce_hbm.at[pl.ds(ce_off + ebase, n)],
                                  cev.at[s, pl.ds(0, n)], gsem[s]).wait()
            pltpu.make_async_copy(db_hbm.at[idxv.at[q, 0]], dbv.at[s],
                                  gsem[s]).wait()
            pltpu.make_async_copy(eh_hbm.at[dst2v.at[q]], ehv.at[s],
                                  gsem[s]).wait()

        def compute(c, s):
            n = nrows_of(c)

            @plsc.parallel_loop(0, n, unroll=4)
            def _crow(r):
                for v in range(HH // 16):
                    sl = pl.ds(v * 16, 16)
                    sl_hi = pl.ds(HH + v * 16, 16)
                    e = cev[s, r, sl] + dbv[s, r, sl] + ehv[s, r, sl]
                    cev[s, r, sl] = e
                    sg = 1.0 / (1.0 + jnp.exp(-e))
                    dbv[s, r, sl] = sg * dbv[s, r, sl_hi]
                    dbv[s, r, sl_hi] = sg
            if n < _EC:
                @plsc.parallel_loop(n, _EC, unroll=4)
                def _ztail(r):
                    for v in range(HID // 16):
                        dbv[s, r, pl.ds(v * 16, 16)] = zero16

        def issue_writes(c, q, s):
            n = nrows_of(c)
            ebase = tbase + c * _EC
            if eraw_hbm is not None:
                pltpu.async_copy(cev.at[s, pl.ds(0, n)],
                                 eraw_hbm.at[pl.ds(ce_off + ebase, n)],
                                 wsem[s])
            pltpu.sync_copy(dbv.at[s], acc_sp.at[idxv.at[q, 1]], add=True)

        def wait_writes(c, q, s):
            n = nrows_of(c)
            ebase = tbase + c * _EC
            if eraw_hbm is not None:
                pltpu.make_async_copy(
                    cev.at[s, pl.ds(0, n)],
                    eraw_hbm.at[pl.ds(ce_off + ebase, n)], wsem[s]).wait()

        def visit(c, cm, first=False, steady=True):
            if steady or c + 2 <= _NCT - 1:
                issue_idx(c + 2, (cm + 2) % _ISLOT)
            if (steady or c >= 2) and not first:
                wait_writes(c - 2, (cm - 2) % _ISLOT, (cm - 2) % _NSLOT)
            if steady or c + 1 <= _NCT - 1:
                prep_gathers(c + 1, (cm + 1) % _ISLOT, (cm + 1) % _NSLOT)
            wait_gathers(c, cm % _ISLOT, cm % _NSLOT)
            compute(c, cm % _NSLOT)
            issue_writes(c, cm % _ISLOT, cm % _NSLOT)

        issue_idx(0, 0)
        issue_idx(1, 1)
        prep_gathers(0, 0, 0)
        visit(0, 0, first=True, steady=False)
        visit(1, 1, first=True, steady=False)

        def steady_body(i, _):
            cb = 2 + i * 6
            for b in range(6):
                visit(cb + b, 2 + b)
            return 0
        lax.fori_loop(0, 68, steady_body, 0)

        for c in range(410, _NCT):
            visit(c, c, steady=False)
        wait_writes(_NCT - 2, (_NCT - 2) % _ISLOT, (_NCT - 2) % _NSLOT)
        wait_writes(_NCT - 1, (_NCT - 1) % _ISLOT, (_NCT - 1) % _NSLOT)

        plsc.subcore_barrier()
        fbase = sid * _NPT
        out_off = cid * _NNP
        pltpu.sync_copy(acc_sp.at[pl.ds(fbase, _NPT)],
                        acc_hbm.at[pl.ds(out_off + fbase, _NPT)])

    return pl.kernel(body, out_type=tuple(outs), mesh=_sc_mesh(),
                     scratch_types=scratch,
                     compiler_params=pltpu.CompilerParams(
                         use_tc_tiling_on_sc=False))


_edge_pass_w = _make_edge_pass(True)
_edge_pass_nw = _make_edge_pass(False)


_TPW = NTRIP // (_NC * _NS)
_TCH = _TPW // _CHUNK


def _make_head_gather():
    outs = (jax.ShapeDtypeStruct((NTRIP, HID), F32),
            jax.ShapeDtypeStruct((NTRIP, HID), F32))
    scratch = [
        pltpu.VMEM((_CHUNK,), jnp.int32),
        pltpu.VMEM((_CHUNK, HID), F32),
    ]

    def body(h_hbm, t0_hbm, t2_hbm, s_hbm, o_hbm, idxv, rowsv):
        cid = lax.axis_index("c")
        sid = lax.axis_index("s")
        base = (sid * _NC + cid) * _TPW
        for t_hbm, out_hbm in ((t0_hbm, s_hbm), (t2_hbm, o_hbm)):
            def cb(c, _, t_hbm=t_hbm, out_hbm=out_hbm):
                rb = base + c * _CHUNK
                pltpu.sync_copy(t_hbm.at[pl.ds(rb, _CHUNK)], idxv)
                pltpu.sync_copy(h_hbm.at[idxv], rowsv)
                pltpu.sync_copy(rowsv, out_hbm.at[pl.ds(rb, _CHUNK)])
                return 0
            lax.fori_loop(0, _TCH, cb, 0)

    return pl.kernel(body, out_type=outs, mesh=_sc_mesh(),
                     scratch_types=scratch)


_head_gather = _make_head_gather()


def _dot(a, b):
    return jnp.dot(a, b, preferred_element_type=F32)


def _proj_node(x, w, b):
    def body(x_ref, w_ref, b_ref, o_ref):
        o_ref[...] = _dot(x_ref[...], w_ref[...]) + b_ref[...]
    return pl.pallas_call(
        body, out_shape=jax.ShapeDtypeStruct((NN, HID), F32),
    )(x, w, b.reshape(1, HID))


_ER = 3200
_EG = NE // _ER


def _proj_edge(et, lw, lb, cw, cb):
    def body(x_ref, lw_ref, lb_ref, cw_ref, cb_ref, e_ref, ce_ref):
        e = _dot(x_ref[...], lw_ref[...]) + lb_ref[...]
        e_ref[0] = e[:, :HH]
        e_ref[1] = e[:, HH:]
        ce = _dot(e, cw_ref[...]) + cb_ref[...]
        ce_ref[0] = ce[:, :HH]
        ce_ref[1] = ce[:, HH:]
    return pl.pallas_call(
        body,
        grid=(_EG,),
        in_specs=[
            pl.BlockSpec((_ER, 16), lambda i: (i, 0)),
            pl.BlockSpec((16, HID), lambda i: (0, 0)),
            pl.BlockSpec((1, HID), lambda i: (0, 0)),
            pl.BlockSpec((HID, HID), lambda i: (0, 0)),
            pl.BlockSpec((1, HID), lambda i: (0, 0)),
        ],
        out_specs=[
            pl.BlockSpec((2, _ER, HH), lambda i: (0, i, 0)),
            pl.BlockSpec((2, _ER, HH), lambda i: (0, i, 0)),
        ],
        out_shape=[
            jax.ShapeDtypeStruct((2, NE, HH), F32),
            jax.ShapeDtypeStruct((2, NE, HH), F32),
        ],
    )(et, lw, lb.reshape(1, HID), cw, cb.reshape(1, HID))


def _node_mm(h, lp):
    def body(h_ref, aw, ab, bw, bb, dw, dbias, ew, eb, ah_ref, dbt_ref, eht_ref):
        hh = h_ref[...]
        Ah = _dot(hh, aw[...]) + ab[...]
        Bh = _dot(hh, bw[...]) + bb[...]
        Dh = _dot(hh, dw[...]) + dbias[...]
        Eh = _dot(hh, ew[...]) + eb[...]
        ah_ref[...] = Ah
        dbt_ref[0] = jnp.concatenate([Dh[:, :HH], Bh[:, :HH]], axis=1)
        dbt_ref[1] = jnp.concatenate([Dh[:, HH:], Bh[:, HH:]], axis=1)
        eht_ref[0] = Eh[:, :HH]
        eht_ref[1] = Eh[:, HH:]
    return pl.pallas_call(
        body,
        out_shape=[
            jax.ShapeDtypeStruct((NN, HID), F32),
            jax.ShapeDtypeStruct((2, NN, HID), F32),
            jax.ShapeDtypeStruct((2, NN, HH), F32),
        ],
    )(h, lp['A_w'], lp['A_b'].reshape(1, HID),
      lp['B_w'], lp['B_b'].reshape(1, HID),
      lp['D_w'], lp['D_b'].reshape(1, HID),
      lp['E_w'], lp['E_b'].reshape(1, HID))


def _h_update(h_in, ah, acc, norm_n, bn_g, bn_b):
    def body(hin_ref, ah_ref, acc_ref, nn_ref, g_ref, b_ref, out_ref):
        num_f = jnp.concatenate([acc_ref[0, :NN, :HH], acc_ref[1, :NN, :HH]],
                                axis=1)
        den_f = jnp.concatenate([acc_ref[0, :NN, HH:], acc_ref[1, :NN, HH:]],
                                axis=1)
        hn = ah_ref[...] + num_f / (den_f + 1e-6)
        hn = hn * nn_ref[...]
        mu = jnp.mean(hn, axis=0, keepdims=True)
        var = jnp.mean((hn - mu) ** 2, axis=0, keepdims=True)
        hn = (hn - mu) * lax.rsqrt(var + 1e-5) * g_ref[...] + b_ref[...]
        out_ref[...] = hin_ref[...] + jnp.maximum(hn, 0.0)
    return pl.pallas_call(
        body, out_shape=jax.ShapeDtypeStruct((NN, HID), F32),
    )(h_in, ah, acc, norm_n, bn_g.reshape(1, HID), bn_b.reshape(1, HID))


def _e_stats(eraw, norm_e):
    def body(er_ref, ne_ref, out_ref):
        i = pl.program_id(0)
        y = jnp.concatenate([er_ref[0], er_ref[1]], axis=1) * ne_ref[...]
        @pl.when(i == 0)
        def _():
            out_ref[...] = jnp.zeros_like(out_ref)
        out_ref[0:1, :] += jnp.sum(y, axis=0, keepdims=True)
        out_ref[1:2, :] += jnp.sum(y * y, axis=0, keepdims=True)
    return pl.pallas_call(
        body,
        grid=(_EG,),
        in_specs=[
            pl.BlockSpec((2, _ER, HH), lambda i: (0, i, 0)),
            pl.BlockSpec((_ER, 1), lambda i: (i, 0)),
        ],
        out_specs=pl.BlockSpec((8, HID), lambda i: (0, 0)),
        out_shape=jax.ShapeDtypeStruct((8, HID), F32),
    )(eraw, norm_e)


def _e_apply(eraw, e_in, norm_e, st, bn_g, bn_b, cw, cb):
    def body(er_ref, ein_ref, ne_ref, st_ref, g_ref, b_ref, cw_ref, cb_ref,
             enew_ref, ce_ref):
        y = jnp.concatenate([er_ref[0], er_ref[1]], axis=1) * ne_ref[...]
        mu = st_ref[0:1, :] * (1.0 / NE)
        var = st_ref[1:2, :] * (1.0 / NE) - mu * mu
        z = (y - mu) * lax.rsqrt(var + 1e-5) * g_ref[...] + b_ref[...]
        z = jnp.maximum(z, 0.0)
        enew = jnp.concatenate([ein_ref[0], ein_ref[1]], axis=1) + z
        enew_ref[0] = enew[:, :HH]
        enew_ref[1] = enew[:, HH:]
        ce = _dot(enew, cw_ref[...]) + cb_ref[...]
        ce_ref[0] = ce[:, :HH]
        ce_ref[1] = ce[:, HH:]
    return pl.pallas_call(
        body,
        grid=(_EG,),
        in_specs=[
            pl.BlockSpec((2, _ER, HH), lambda i: (0, i, 0)),
            pl.BlockSpec((2, _ER, HH), lambda i: (0, i, 0)),
            pl.BlockSpec((_ER, 1), lambda i: (i, 0)),
            pl.BlockSpec((8, HID), lambda i: (0, 0)),
            pl.BlockSpec((1, HID), lambda i: (0, 0)),
            pl.BlockSpec((1, HID), lambda i: (0, 0)),
            pl.BlockSpec((HID, HID), lambda i: (0, 0)),
            pl.BlockSpec((1, HID), lambda i: (0, 0)),
        ],
        out_specs=[
            pl.BlockSpec((2, _ER, HH), lambda i: (0, i, 0)),
            pl.BlockSpec((2, _ER, HH), lambda i: (0, i, 0)),
        ],
        out_shape=[
            jax.ShapeDtypeStruct((2, NE, HH), F32),
            jax.ShapeDtypeStruct((2, NE, HH), F32),
        ],
    )(eraw, e_in, norm_e, st, bn_g.reshape(1, HID), bn_b.reshape(1, HID),
      cw, cb.reshape(1, HID))


_HR = 4096
_HG = NTRIP // _HR


def _head_mlp(s, o, fc1_w, fc1_b, bn1_g, bn1_b, out_w, out_b):
    def body1(s_ref, o_ref, w1s_ref, w1o_ref, b1_ref, f_ref, st_ref):
        i = pl.program_id(0)
        f = (_dot(s_ref[...], w1s_ref[...]) + _dot(o_ref[...], w1o_ref[...])
             + b1_ref[...])
        f_ref[...] = f
        @pl.when(i == 0)
        def _():
            st_ref[...] = jnp.zeros_like(st_ref)
        st_ref[0:1, :] += jnp.sum(f, axis=0, keepdims=True)
        st_ref[1:2, :] += jnp.sum(f * f, axis=0, keepdims=True)

    f, st = pl.pallas_call(
        body1,
        grid=(_HG,),
        in_specs=[
            pl.BlockSpec((_HR, HID), lambda i: (i, 0)),
            pl.BlockSpec((_HR, HID), lambda i: (i, 0)),
            pl.BlockSpec((HID, 200), lambda i: (0, 0)),
            pl.BlockSpec((HID, 200), lambda i: (0, 0)),
            pl.BlockSpec((1, 200), lambda i: (0, 0)),
        ],
        out_specs=[
            pl.BlockSpec((_HR, 200), lambda i: (i, 0)),
            pl.BlockSpec((8, 200), lambda i: (0, 0)),
        ],
        out_shape=[
            jax.ShapeDtypeStruct((NTRIP, 200), F32),
            jax.ShapeDtypeStruct((8, 200), F32),
        ],
    )(s, o, fc1_w[:HID], fc1_w[HID:], fc1_b.reshape(1, -1))

    def body2(f_ref, st_ref, g_ref, b_ref, ow_ref, ob_ref, out_ref):
        mu = st_ref[0:1, :] * (1.0 / NTRIP)
        var = st_ref[1:2, :] * (1.0 / NTRIP) - mu * mu
        z = (f_ref[...] - mu) * lax.rsqrt(var + 1e-5) * g_ref[...] + b_ref[...]
        z = jnp.maximum(z, 0.0)
        out_ref[...] = _dot(z, ow_ref[...]) + ob_ref[...]

    return pl.pallas_call(
        body2,
        grid=(_HG,),
        in_specs=[
            pl.BlockSpec((_HR, 200), lambda i: (i, 0)),
            pl.BlockSpec((8, 200), lambda i: (0, 0)),
            pl.BlockSpec((1, 200), lambda i: (0, 0)),
            pl.BlockSpec((1, 200), lambda i: (0, 0)),
            pl.BlockSpec((200, 1), lambda i: (0, 0)),
            pl.BlockSpec((1, 1), lambda i: (0, 0)),
        ],
        out_specs=pl.BlockSpec((_HR, 1), lambda i: (i, 0)),
        out_shape=jax.ShapeDtypeStruct((NTRIP, 1), F32),
    )(f, st, bn1_g.reshape(1, -1), bn1_b.reshape(1, -1), out_w,
      out_b.reshape(1, 1))


def kernel(node_id, edge_type, norm_n, norm_e, params, g, triplets):
    p = params
    t0 = triplets[:, 0]
    t2 = triplets[:, 2]

    h = _proj_node(node_id, p['lh_w'], p['lh_b'])
    l0 = p['layers'][0]
    e, ce = _proj_edge(edge_type, p['le_w'], p['le_b'], l0['C_w'], l0['C_b'])

    for li in range(len(p['layers'])):
        lp = p['layers'][li]
        ah, dbt, eht = _node_mm(h, lp)
        dbt_f = dbt.reshape(_NC * NN, HID)
        eht_f = eht.reshape(_NC * NN, HH)
        ce_f = ce.reshape(_NC * NE, HH)
        last = li == len(p['layers']) - 1
        if last:
            acc = _edge_pass_nw(ce_f, dbt_f, eht_f, g)
            if isinstance(acc, (tuple, list)):
                acc = acc[0]
            eraw = None
        else:
            acc, eraw = _edge_pass_w(ce_f, dbt_f, eht_f, g)
            eraw = eraw.reshape(2, NE, HH)
        h = _h_update(h, ah, acc.reshape(2, _NNP, HID),
                      norm_n, lp['bn_h_g'], lp['bn_h_b'])
        if not last:
            nlp = p['layers'][li + 1]
            st = _e_stats(eraw, norm_e)
            e, ce = _e_apply(eraw, e, norm_e, st, lp['bn_e_g'], lp['bn_e_b'],
                             nlp['C_w'], nlp['C_b'])

    s, o = _head_gather(h, t0, t2)
    out = _head_mlp(s, o, p['fc1_w'], p['fc1_b'], p['bn1_g'], p['bn1_b'],
                    p['out_w'], p['out_b'])
    return (h, out)


# speedup vs baseline: 4.3172x; 1.6263x over previous
import jax
import jax.numpy as jnp
from jax import lax
from jax.experimental import pallas as pl
from jax.experimental.pallas import tpu as pltpu
from jax.experimental.pallas import tpu_sc as plsc

NN = 10000
NE = 320000
HID = 128
HH = 64
NTRIP = 32768
F32 = jnp.float32

_NC, _NS = 2, 16
_CHUNK = 128
_EPT = NE // _NS
_NFULL = _EPT // _CHUNK
_REM = _EPT - _NFULL * _CHUNK
_NNP = 10240
_NPT = _NNP // _NS


def _sc_mesh():
    return plsc.VectorSubcoreMesh(
        core_axis_name="c", subcore_axis_name="s",
        num_cores=_NC, num_subcores=_NS)


_EC = 48
_ENF = _EPT // _EC
_EREM = _EPT - _ENF * _EC
_NCT = _ENF + 1
_NSLOT = 3
_ISLOT = 6


def _make_edge_pass(write_eraw: bool):
    outs = [
        jax.ShapeDtypeStruct((_NC * _NNP, HID), F32),
    ]
    if write_eraw:
        outs.append(jax.ShapeDtypeStruct((NE, HID), F32))
    scratch = [
        pltpu.VMEM((_ISLOT, 2, _EC), jnp.int32),
        pltpu.VMEM((_ISLOT, _EC), jnp.int32),
        pltpu.VMEM((_NSLOT, _EC, HH), F32),
        pltpu.VMEM((_NSLOT, _EC, HID), F32),
        pltpu.VMEM((_NSLOT, _EC, HH), F32),
        pltpu.VMEM_SHARED((_NNP, HID), F32),
    ] + [pltpu.SemaphoreType.DMA] * (2 * _NSLOT + _ISLOT)

    def body(ce_hbm, db_hbm, eh_hbm, g_hbm, *rest):
        if write_eraw:
            (acc_hbm, eraw_hbm, idxv, dst2v, cev, dbv, ehv, acc_sp,
             *sems) = rest
        else:
            (acc_hbm, idxv, dst2v, cev, dbv, ehv, acc_sp, *sems) = rest
            eraw_hbm = None
        gsem = sems[:_NSLOT]
        wsem = sems[_NSLOT:2 * _NSLOT]
        isem = sems[2 * _NSLOT:]

        cid = lax.axis_index("c")
        sid = lax.axis_index("s")
        zero16 = jnp.zeros((16,), F32)
        row_off = cid * NN
        col_off = cid * HH

        @plsc.parallel_loop(0, _EC, unroll=4)
        def _zrow(r):
            for v in range(HID // 16):
                dbv[0, r, pl.ds(v * 16, 16)] = zero16
        zbase = sid * _NPT
        for nr, qo in [(_EC, 48 * q) for q in range(13)] + [(16, 624)]:
            pltpu.sync_copy(dbv.at[0, pl.ds(0, nr)],
                            acc_sp.at[pl.ds(zbase + qo, nr)])
        plsc.subcore_barrier()

        tbase = sid * _EPT

        def nrows_of(c):
            return _EREM if (isinstance(c, int) and c == _NCT - 1) else _EC

        def issue_idx(c, q):
            n = nrows_of(c)
            ebase = tbase + c * _EC
            pltpu.async_copy(g_hbm.at[:, pl.ds(ebase, n)],
                             idxv.at[q, :, pl.ds(0, n)], isem[q])

        def prep_gathers(c, q, s):
            n = nrows_of(c)
            ebase = tbase + c * _EC
            pltpu.make_async_copy(g_hbm.at[:, pl.ds(ebase, n)],
                                  idxv.at[q, :, pl.ds(0, n)], isem[q]).wait()
            off_vec = jnp.zeros((16,), jnp.int32) + row_off
            for v in range(_EC // 16):
                sl = pl.ds(v * 16, 16)
                if v * 16 < n:
                    idxv[q, 0, sl] = idxv[q, 0, sl] + row_off
                    dst2v[q, sl] = idxv[q, 1, sl] + row_off
                else:
                    idxv[q, 0, sl] = off_vec
                    dst2v[q, sl] = off_vec
            pltpu.async_copy(ce_hbm.at[pl.ds(ebase, n), pl.ds(col_off, HH)],
                             cev.at[s, pl.ds(0, n)], gsem[s])
            pltpu.async_copy(db_hbm.at[idxv.at[q, 0]], dbv.at[s], gsem[s])
            pltpu.async_copy(eh_hbm.at[dst2v.at[q]], ehv.at[s], gsem[s])

        def wait_gathers(c, q, s):
            n = nrows_of(c)
            ebase = tbase + c * _EC
            pltpu.make_async_copy(ce_hbm.at[pl.ds(ebase, n),
                                            pl.ds(col_off, HH)],
                                  cev.at[s, pl.ds(0, n)], gsem[s]).wait()
            pltpu.make_async_copy(db_hbm.at[idxv.at[q, 0]], dbv.at[s],
                                  gsem[s]).wait()
            pltpu.make_async_copy(eh_hbm.at[dst2v.at[q]], ehv.at[s],
                                  gsem[s]).wait()

        def compute(c, s):
            n = nrows_of(c)

            @plsc.parallel_loop(0, n, unroll=4)
            def _crow(r):
                for v in range(HH // 16):
                    sl = pl.ds(v * 16, 16)
                    sl_hi = pl.ds(HH + v * 16, 16)
                    e = cev[s, r, sl] + dbv[s, r, sl] + ehv[s, r, sl]
                    cev[s, r, sl] = e
                    sg = 1.0 / (1.0 + jnp.exp(-e))
                    dbv[s, r, sl] = sg * dbv[s, r, sl_hi]
                    dbv[s, r, sl_hi] = sg
            if n < _EC:
                @plsc.parallel_loop(n, _EC, unroll=4)
                def _ztail(r):
                    for v in range(HID // 16):
                        dbv[s, r, pl.ds(v * 16, 16)] = zero16

        def issue_writes(c, q, s):
            n = nrows_of(c)
            ebase = tbase + c * _EC
            if eraw_hbm is not None:
                pltpu.async_copy(cev.at[s, pl.ds(0, n)],
                                 eraw_hbm.at[pl.ds(ebase, n),
                                             pl.ds(col_off, HH)],
                                 wsem[s])
            pltpu.sync_copy(dbv.at[s], acc_sp.at[idxv.at[q, 1]], add=True)

        def wait_writes(c, q, s):
            n = nrows_of(c)
            ebase = tbase + c * _EC
            if eraw_hbm is not None:
                pltpu.make_async_copy(
                    cev.at[s, pl.ds(0, n)],
                    eraw_hbm.at[pl.ds(ebase, n), pl.ds(col_off, HH)],
                    wsem[s]).wait()

        def visit(c, cm, first=False, steady=True):
            if steady or c + 2 <= _NCT - 1:
                issue_idx(c + 2, (cm + 2) % _ISLOT)
            if (steady or c >= 2) and not first:
                wait_writes(c - 2, (cm - 2) % _ISLOT, (cm - 2) % _NSLOT)
            if steady or c + 1 <= _NCT - 1:
                prep_gathers(c + 1, (cm + 1) % _ISLOT, (cm + 1) % _NSLOT)
            wait_gathers(c, cm % _ISLOT, cm % _NSLOT)
            compute(c, cm % _NSLOT)
            issue_writes(c, cm % _ISLOT, cm % _NSLOT)

        issue_idx(0, 0)
        issue_idx(1, 1)
        prep_gathers(0, 0, 0)
        visit(0, 0, first=True, steady=False)
        visit(1, 1, first=True, steady=False)

        def steady_body(i, _):
            cb = 2 + i * 6
            for b in range(6):
                visit(cb + b, 2 + b)
            return 0
        lax.fori_loop(0, 68, steady_body, 0)

        for c in range(410, _NCT):
            visit(c, c, steady=False)
        wait_writes(_NCT - 2, (_NCT - 2) % _ISLOT, (_NCT - 2) % _NSLOT)
        wait_writes(_NCT - 1, (_NCT - 1) % _ISLOT, (_NCT - 1) % _NSLOT)

        plsc.subcore_barrier()
        fbase = sid * _NPT
        out_off = cid * _NNP
        pltpu.sync_copy(acc_sp.at[pl.ds(fbase, _NPT)],
                        acc_hbm.at[pl.ds(out_off + fbase, _NPT)])

    return pl.kernel(body, out_type=tuple(outs), mesh=_sc_mesh(),
                     scratch_types=scratch,
                     compiler_params=pltpu.CompilerParams(
                         use_tc_tiling_on_sc=False))


_edge_pass_w = _make_edge_pass(True)
_edge_pass_nw = _make_edge_pass(False)


_TPW = NTRIP // (_NC * _NS)
_TCH = _TPW // _CHUNK


def _make_head_gather():
    outs = (jax.ShapeDtypeStruct((NTRIP, HID), F32),
            jax.ShapeDtypeStruct((NTRIP, HID), F32))
    scratch = [
        pltpu.VMEM((_CHUNK,), jnp.int32),
        pltpu.VMEM((_CHUNK, HID), F32),
    ]

    def body(h_hbm, t0_hbm, t2_hbm, s_hbm, o_hbm, idxv, rowsv):
        cid = lax.axis_index("c")
        sid = lax.axis_index("s")
        base = (sid * _NC + cid) * _TPW
        for t_hbm, out_hbm in ((t0_hbm, s_hbm), (t2_hbm, o_hbm)):
            def cb(c, _, t_hbm=t_hbm, out_hbm=out_hbm):
                rb = base + c * _CHUNK
                pltpu.sync_copy(t_hbm.at[pl.ds(rb, _CHUNK)], idxv)
                pltpu.sync_copy(h_hbm.at[idxv], rowsv)
                pltpu.sync_copy(rowsv, out_hbm.at[pl.ds(rb, _CHUNK)])
                return 0
            lax.fori_loop(0, _TCH, cb, 0)

    return pl.kernel(body, out_type=outs, mesh=_sc_mesh(),
                     scratch_types=scratch)


_head_gather = _make_head_gather()


def _dot(a, b):
    return jnp.dot(a, b, preferred_element_type=F32)


def _proj_node(x, w, b):
    def body(x_ref, w_ref, b_ref, o_ref):
        o_ref[...] = _dot(x_ref[...], w_ref[...]) + b_ref[...]
    return pl.pallas_call(
        body, out_shape=jax.ShapeDtypeStruct((NN, HID), F32),
    )(x, w, b.reshape(1, HID))


_ER = 3200
_EG = NE // _ER


def _proj_edge(et, lw, lb, cw, cb):
    def body(x_ref, lw_ref, lb_ref, cw_ref, cb_ref, e_ref, ce_ref):
        e = _dot(x_ref[...], lw_ref[...]) + lb_ref[...]
        e_ref[...] = e
        ce_ref[...] = _dot(e, cw_ref[...]) + cb_ref[...]
    return pl.pallas_call(
        body,
        grid=(_EG,),
        in_specs=[
            pl.BlockSpec((_ER, 16), lambda i: (i, 0)),
            pl.BlockSpec((16, HID), lambda i: (0, 0)),
            pl.BlockSpec((1, HID), lambda i: (0, 0)),
            pl.BlockSpec((HID, HID), lambda i: (0, 0)),
            pl.BlockSpec((1, HID), lambda i: (0, 0)),
        ],
        out_specs=[
            pl.BlockSpec((_ER, HID), lambda i: (i, 0)),
            pl.BlockSpec((_ER, HID), lambda i: (i, 0)),
        ],
        out_shape=[
            jax.ShapeDtypeStruct((NE, HID), F32),
            jax.ShapeDtypeStruct((NE, HID), F32),
        ],
    )(et, lw, lb.reshape(1, HID), cw, cb.reshape(1, HID))


def _node_mm(h, lp):
    def body(h_ref, aw, ab, bw, bb, dw, dbias, ew, eb, ah_ref, dbt_ref, eht_ref):
        hh = h_ref[...]
        Ah = _dot(hh, aw[...]) + ab[...]
        Bh = _dot(hh, bw[...]) + bb[...]
        Dh = _dot(hh, dw[...]) + dbias[...]
        Eh = _dot(hh, ew[...]) + eb[...]
        ah_ref[...] = Ah
        dbt_ref[0] = jnp.concatenate([Dh[:, :HH], Bh[:, :HH]], axis=1)
        dbt_ref[1] = jnp.concatenate([Dh[:, HH:], Bh[:, HH:]], axis=1)
        eht_ref[0] = Eh[:, :HH]
        eht_ref[1] = Eh[:, HH:]
    return pl.pallas_call(
        body,
        out_shape=[
            jax.ShapeDtypeStruct((NN, HID), F32),
            jax.ShapeDtypeStruct((2, NN, HID), F32),
            jax.ShapeDtypeStruct((2, NN, HH), F32),
        ],
    )(h, lp['A_w'], lp['A_b'].reshape(1, HID),
      lp['B_w'], lp['B_b'].reshape(1, HID),
      lp['D_w'], lp['D_b'].reshape(1, HID),
      lp['E_w'], lp['E_b'].reshape(1, HID))


def _h_update(h_in, ah, acc, norm_n, bn_g, bn_b):
    def body(hin_ref, ah_ref, acc_ref, nn_ref, g_ref, b_ref, out_ref):
        num_f = jnp.concatenate([acc_ref[0, :NN, :HH], acc_ref[1, :NN, :HH]],
                                axis=1)
        den_f = jnp.concatenate([acc_ref[0, :NN, HH:], acc_ref[1, :NN, HH:]],
                                axis=1)
        hn = ah_ref[...] + num_f / (den_f + 1e-6)
        hn = hn * nn_ref[...]
        mu = jnp.mean(hn, axis=0, keepdims=True)
        var = jnp.mean((hn - mu) ** 2, axis=0, keepdims=True)
        hn = (hn - mu) * lax.rsqrt(var + 1e-5) * g_ref[...] + b_ref[...]
        out_ref[...] = hin_ref[...] + jnp.maximum(hn, 0.0)
    return pl.pallas_call(
        body, out_shape=jax.ShapeDtypeStruct((NN, HID), F32),
    )(h_in, ah, acc, norm_n, bn_g.reshape(1, HID), bn_b.reshape(1, HID))


def _e_stats(eraw, norm_e):
    def body(er_ref, ne_ref, out_ref):
        i = pl.program_id(0)
        y = er_ref[...] * ne_ref[...]
        @pl.when(i == 0)
        def _():
            out_ref[...] = jnp.zeros_like(out_ref)
        out_ref[0:1, :] += jnp.sum(y, axis=0, keepdims=True)
        out_ref[1:2, :] += jnp.sum(y * y, axis=0, keepdims=True)
    return pl.pallas_call(
        body,
        grid=(_EG,),
        in_specs=[
            pl.BlockSpec((_ER, HID), lambda i: (i, 0)),
            pl.BlockSpec((_ER, 1), lambda i: (i, 0)),
        ],
        out_specs=pl.BlockSpec((8, HID), lambda i: (0, 0)),
        out_shape=jax.ShapeDtypeStruct((8, HID), F32),
    )(eraw, norm_e)


def _e_apply(eraw, e_in, norm_e, st, bn_g, bn_b, cw, cb):
    def body(er_ref, ein_ref, ne_ref, st_ref, g_ref, b_ref, cw_ref, cb_ref,
             enew_ref, ce_ref):
        y = er_ref[...] * ne_ref[...]
        mu = st_ref[0:1, :] * (1.0 / NE)
        var = st_ref[1:2, :] * (1.0 / NE) - mu * mu
        z = (y - mu) * lax.rsqrt(var + 1e-5) * g_ref[...] + b_ref[...]
        z = jnp.maximum(z, 0.0)
        enew = ein_ref[...] + z
        enew_ref[...] = enew
        ce_ref[...] = _dot(enew, cw_ref[...]) + cb_ref[...]
    return pl.pallas_call(
        body,
        grid=(_EG,),
        in_specs=[
            pl.BlockSpec((_ER, HID), lambda i: (i, 0)),
            pl.BlockSpec((_ER, HID), lambda i: (i, 0)),
            pl.BlockSpec((_ER, 1), lambda i: (i, 0)),
            pl.BlockSpec((8, HID), lambda i: (0, 0)),
            pl.BlockSpec((1, HID), lambda i: (0, 0)),
            pl.BlockSpec((1, HID), lambda i: (0, 0)),
            pl.BlockSpec((HID, HID), lambda i: (0, 0)),
            pl.BlockSpec((1, HID), lambda i: (0, 0)),
        ],
        out_specs=[
            pl.BlockSpec((_ER, HID), lambda i: (i, 0)),
            pl.BlockSpec((_ER, HID), lambda i: (i, 0)),
        ],
        out_shape=[
            jax.ShapeDtypeStruct((NE, HID), F32),
            jax.ShapeDtypeStruct((NE, HID), F32),
        ],
    )(eraw, e_in, norm_e, st, bn_g.reshape(1, HID), bn_b.reshape(1, HID),
      cw, cb.reshape(1, HID))


_HR = 4096
_HG = NTRIP // _HR


def _head_mlp(s, o, fc1_w, fc1_b, bn1_g, bn1_b, out_w, out_b):
    def body1(s_ref, o_ref, w1s_ref, w1o_ref, b1_ref, f_ref, st_ref):
        i = pl.program_id(0)
        f = (_dot(s_ref[...], w1s_ref[...]) + _dot(o_ref[...], w1o_ref[...])
             + b1_ref[...])
        f_ref[...] = f
        @pl.when(i == 0)
        def _():
            st_ref[...] = jnp.zeros_like(st_ref)
        st_ref[0:1, :] += jnp.sum(f, axis=0, keepdims=True)
        st_ref[1:2, :] += jnp.sum(f * f, axis=0, keepdims=True)

    f, st = pl.pallas_call(
        body1,
        grid=(_HG,),
        in_specs=[
            pl.BlockSpec((_HR, HID), lambda i: (i, 0)),
            pl.BlockSpec((_HR, HID), lambda i: (i, 0)),
            pl.BlockSpec((HID, 200), lambda i: (0, 0)),
            pl.BlockSpec((HID, 200), lambda i: (0, 0)),
            pl.BlockSpec((1, 200), lambda i: (0, 0)),
        ],
        out_specs=[
            pl.BlockSpec((_HR, 200), lambda i: (i, 0)),
            pl.BlockSpec((8, 200), lambda i: (0, 0)),
        ],
        out_shape=[
            jax.ShapeDtypeStruct((NTRIP, 200), F32),
            jax.ShapeDtypeStruct((8, 200), F32),
        ],
    )(s, o, fc1_w[:HID], fc1_w[HID:], fc1_b.reshape(1, -1))

    def body2(f_ref, st_ref, g_ref, b_ref, ow_ref, ob_ref, out_ref):
        mu = st_ref[0:1, :] * (1.0 / NTRIP)
        var = st_ref[1:2, :] * (1.0 / NTRIP) - mu * mu
        z = (f_ref[...] - mu) * lax.rsqrt(var + 1e-5) * g_ref[...] + b_ref[...]
        z = jnp.maximum(z, 0.0)
        out_ref[...] = _dot(z, ow_ref[...]) + ob_ref[...]

    return pl.pallas_call(
        body2,
        grid=(_HG,),
        in_specs=[
            pl.BlockSpec((_HR, 200), lambda i: (i, 0)),
            pl.BlockSpec((8, 200), lambda i: (0, 0)),
            pl.BlockSpec((1, 200), lambda i: (0, 0)),
            pl.BlockSpec((1, 200), lambda i: (0, 0)),
            pl.BlockSpec((200, 1), lambda i: (0, 0)),
            pl.BlockSpec((1, 1), lambda i: (0, 0)),
        ],
        out_specs=pl.BlockSpec((_HR, 1), lambda i: (i, 0)),
        out_shape=jax.ShapeDtypeStruct((NTRIP, 1), F32),
    )(f, st, bn1_g.reshape(1, -1), bn1_b.reshape(1, -1), out_w,
      out_b.reshape(1, 1))


def kernel(node_id, edge_type, norm_n, norm_e, params, g, triplets):
    p = params
    t0 = triplets[:, 0]
    t2 = triplets[:, 2]

    h = _proj_node(node_id, p['lh_w'], p['lh_b'])
    l0 = p['layers'][0]
    e, ce = _proj_edge(edge_type, p['le_w'], p['le_b'], l0['C_w'], l0['C_b'])

    for li in range(len(p['layers'])):
        lp = p['layers'][li]
        ah, dbt, eht = _node_mm(h, lp)
        dbt_f = dbt.reshape(_NC * NN, HID)
        eht_f = eht.reshape(_NC * NN, HH)
        last = li == len(p['layers']) - 1
        if last:
            acc = _edge_pass_nw(ce, dbt_f, eht_f, g)
            if isinstance(acc, (tuple, list)):
                acc = acc[0]
            eraw = None
        else:
            acc, eraw = _edge_pass_w(ce, dbt_f, eht_f, g)
        h = _h_update(h, ah, acc.reshape(2, _NNP, HID),
                      norm_n, lp['bn_h_g'], lp['bn_h_b'])
        if not last:
            nlp = p['layers'][li + 1]
            st = _e_stats(eraw, norm_e)
            e, ce = _e_apply(eraw, e, norm_e, st, lp['bn_e_g'], lp['bn_e_b'],
                             nlp['C_w'], nlp['C_b'])

    s, o = _head_gather(h, t0, t2)
    out = _head_mlp(s, o, p['fc1_w'], p['fc1_b'], p['bn1_g'], p['bn1_b'],
                    p['out_w'], p['out_b'])
    return (h, out)


# speedup vs baseline: 4.4438x; 1.0293x over previous
import jax
import jax.numpy as jnp
from jax import lax
from jax.experimental import pallas as pl
from jax.experimental.pallas import tpu as pltpu
from jax.experimental.pallas import tpu_sc as plsc

NN = 10000
NE = 320000
HID = 128
HH = 64
NTRIP = 32768
F32 = jnp.float32

_NC, _NS = 2, 16
_CHUNK = 128
_EPT = NE // _NS
_NFULL = _EPT // _CHUNK
_REM = _EPT - _NFULL * _CHUNK
_NNP = 10240
_NPT = _NNP // _NS


def _sc_mesh():
    return plsc.VectorSubcoreMesh(
        core_axis_name="c", subcore_axis_name="s",
        num_cores=_NC, num_subcores=_NS)


_EC = 48
_ENF = _EPT // _EC
_EREM = _EPT - _ENF * _EC
_NCT = _ENF + 1
_NSLOT = 3
_ISLOT = 6


def _make_edge_pass(write_eraw: bool):
    outs = [
        jax.ShapeDtypeStruct((_NC * _NNP, HID), F32),
    ]
    if write_eraw:
        outs.append(jax.ShapeDtypeStruct((NE, HID), F32))
    scratch = [
        pltpu.VMEM((_ISLOT, 2, _EC), jnp.int32),
        pltpu.VMEM((_ISLOT, _EC), jnp.int32),
        pltpu.VMEM((_NSLOT, _EC, HH), F32),
        pltpu.VMEM((_NSLOT, _EC, HID), F32),
        pltpu.VMEM((_NSLOT, _EC, HH), F32),
        pltpu.VMEM_SHARED((_NNP, HID), F32),
    ] + [pltpu.SemaphoreType.DMA] * (2 * _NSLOT + _ISLOT)

    def body(ce_hbm, db_hbm, eh_hbm, g_hbm, *rest):
        if write_eraw:
            (acc_hbm, eraw_hbm, idxv, dst2v, cev, dbv, ehv, acc_sp,
             *sems) = rest
        else:
            (acc_hbm, idxv, dst2v, cev, dbv, ehv, acc_sp, *sems) = rest
            eraw_hbm = None
        gsem = sems[:_NSLOT]
        wsem = sems[_NSLOT:2 * _NSLOT]
        isem = sems[2 * _NSLOT:]

        cid = lax.axis_index("c")
        sid = lax.axis_index("s")
        zero16 = jnp.zeros((16,), F32)
        row_off = cid * NN
        col_off = cid * HH

        @plsc.parallel_loop(0, _EC, unroll=4)
        def _zrow(r):
            for v in range(HID // 16):
                dbv[0, r, pl.ds(v * 16, 16)] = zero16
        zbase = sid * _NPT
        for nr, qo in [(_EC, 48 * q) for q in range(13)] + [(16, 624)]:
            pltpu.sync_copy(dbv.at[0, pl.ds(0, nr)],
                            acc_sp.at[pl.ds(zbase + qo, nr)])
        plsc.subcore_barrier()

        tbase = sid * _EPT

        def nrows_of(c):
            return _EREM if (isinstance(c, int) and c == _NCT - 1) else _EC

        def issue_idx(c, q):
            n = nrows_of(c)
            ebase = tbase + c * _EC
            pltpu.async_copy(g_hbm.at[:, pl.ds(ebase, n)],
                             idxv.at[q, :, pl.ds(0, n)], isem[q])

        def prep_gathers(c, q, s):
            n = nrows_of(c)
            ebase = tbase + c * _EC
            pltpu.make_async_copy(g_hbm.at[:, pl.ds(ebase, n)],
                                  idxv.at[q, :, pl.ds(0, n)], isem[q]).wait()
            off_vec = jnp.zeros((16,), jnp.int32) + row_off
            for v in range(_EC // 16):
                sl = pl.ds(v * 16, 16)
                if v * 16 < n:
                    idxv[q, 0, sl] = idxv[q, 0, sl] + row_off
                    dst2v[q, sl] = idxv[q, 1, sl] + row_off
                else:
                    idxv[q, 0, sl] = off_vec
                    dst2v[q, sl] = off_vec
            pltpu.async_copy(ce_hbm.at[pl.ds(ebase, n), pl.ds(col_off, HH)],
                             cev.at[s, pl.ds(0, n)], gsem[s])
            pltpu.async_copy(db_hbm.at[idxv.at[q, 0]], dbv.at[s], gsem[s])
            pltpu.async_copy(eh_hbm.at[dst2v.at[q]], ehv.at[s], gsem[s])

        def wait_gathers(c, q, s):
            n = nrows_of(c)
            ebase = tbase + c * _EC
            pltpu.make_async_copy(ce_hbm.at[pl.ds(ebase, n),
                                            pl.ds(col_off, HH)],
                                  cev.at[s, pl.ds(0, n)], gsem[s]).wait()
            pltpu.make_async_copy(db_hbm.at[idxv.at[q, 0]], dbv.at[s],
                                  gsem[s]).wait()
            pltpu.make_async_copy(eh_hbm.at[dst2v.at[q]], ehv.at[s],
                                  gsem[s]).wait()

        def compute(c, s):
            n = nrows_of(c)

            @plsc.parallel_loop(0, n, unroll=4)
            def _crow(r):
                for v in range(HH // 16):
                    sl = pl.ds(v * 16, 16)
                    sl_hi = pl.ds(HH + v * 16, 16)
                    e = cev[s, r, sl] + dbv[s, r, sl] + ehv[s, r, sl]
                    cev[s, r, sl] = e
                    sg = 1.0 / (1.0 + jnp.exp(-e))
                    dbv[s, r, sl] = sg * dbv[s, r, sl_hi]
                    dbv[s, r, sl_hi] = sg
            if n < _EC:
                @plsc.parallel_loop(n, _EC, unroll=4)
                def _ztail(r):
                    for v in range(HID // 16):
                        dbv[s, r, pl.ds(v * 16, 16)] = zero16

        def issue_writes(c, q, s):
            n = nrows_of(c)
            ebase = tbase + c * _EC
            if eraw_hbm is not None:
                pltpu.async_copy(cev.at[s, pl.ds(0, n)],
                                 eraw_hbm.at[pl.ds(ebase, n),
                                             pl.ds(col_off, HH)],
                                 wsem[s])
            pltpu.sync_copy(dbv.at[s], acc_sp.at[idxv.at[q, 1]], add=True)

        def wait_writes(c, q, s):
            n = nrows_of(c)
            ebase = tbase + c * _EC
            if eraw_hbm is not None:
                pltpu.make_async_copy(
                    cev.at[s, pl.ds(0, n)],
                    eraw_hbm.at[pl.ds(ebase, n), pl.ds(col_off, HH)],
                    wsem[s]).wait()

        def visit(c, cm, first=False, steady=True):
            if steady or c + 2 <= _NCT - 1:
                issue_idx(c + 2, (cm + 2) % _ISLOT)
            if (steady or c >= 2) and not first:
                wait_writes(c - 2, (cm - 2) % _ISLOT, (cm - 2) % _NSLOT)
            if steady or c + 1 <= _NCT - 1:
                prep_gathers(c + 1, (cm + 1) % _ISLOT, (cm + 1) % _NSLOT)
            wait_gathers(c, cm % _ISLOT, cm % _NSLOT)
            compute(c, cm % _NSLOT)
            issue_writes(c, cm % _ISLOT, cm % _NSLOT)

        issue_idx(0, 0)
        issue_idx(1, 1)
        prep_gathers(0, 0, 0)
        visit(0, 0, first=True, steady=False)
        visit(1, 1, first=True, steady=False)

        def steady_body(i, _):
            cb = 2 + i * 6
            for b in range(6):
                visit(cb + b, 2 + b)
            return 0
        lax.fori_loop(0, 68, steady_body, 0)

        for c in range(410, _NCT):
            visit(c, c, steady=False)
        wait_writes(_NCT - 2, (_NCT - 2) % _ISLOT, (_NCT - 2) % _NSLOT)
        wait_writes(_NCT - 1, (_NCT - 1) % _ISLOT, (_NCT - 1) % _NSLOT)

        plsc.subcore_barrier()
        fbase = sid * _NPT
        out_off = cid * _NNP
        pltpu.sync_copy(acc_sp.at[pl.ds(fbase, _NPT)],
                        acc_hbm.at[pl.ds(out_off + fbase, _NPT)])

    return pl.kernel(body, out_type=tuple(outs), mesh=_sc_mesh(),
                     scratch_types=scratch,
                     compiler_params=pltpu.CompilerParams(
                         use_tc_tiling_on_sc=False))


_edge_pass_w = _make_edge_pass(True)
_edge_pass_nw = _make_edge_pass(False)


_TPW = NTRIP // (_NC * _NS)
_TCH = _TPW // _CHUNK


def _make_head_gather():
    outs = (jax.ShapeDtypeStruct((NTRIP, HID), F32),
            jax.ShapeDtypeStruct((NTRIP, HID), F32))
    scratch = [
        pltpu.VMEM((_CHUNK,), jnp.int32),
        pltpu.VMEM((_CHUNK, HID), F32),
    ]

    def body(h_hbm, t0_hbm, t2_hbm, s_hbm, o_hbm, idxv, rowsv):
        cid = lax.axis_index("c")
        sid = lax.axis_index("s")
        base = (sid * _NC + cid) * _TPW
        for t_hbm, out_hbm in ((t0_hbm, s_hbm), (t2_hbm, o_hbm)):
            def cb(c, _, t_hbm=t_hbm, out_hbm=out_hbm):
                rb = base + c * _CHUNK
                pltpu.sync_copy(t_hbm.at[pl.ds(rb, _CHUNK)], idxv)
                pltpu.sync_copy(h_hbm.at[idxv], rowsv)
                pltpu.sync_copy(rowsv, out_hbm.at[pl.ds(rb, _CHUNK)])
                return 0
            lax.fori_loop(0, _TCH, cb, 0)

    return pl.kernel(body, out_type=outs, mesh=_sc_mesh(),
                     scratch_types=scratch)


_head_gather = _make_head_gather()


def _dot(a, b):
    return jnp.dot(a, b, preferred_element_type=F32)


def _proj_node(x, w, b):
    def body(x_ref, w_ref, b_ref, o_ref):
        o_ref[...] = _dot(x_ref[...], w_ref[...]) + b_ref[...]
    return pl.pallas_call(
        body, out_shape=jax.ShapeDtypeStruct((NN, HID), F32),
    )(x, w, b.reshape(1, HID))


_ER = 6400
_EG = NE // _ER


def _proj_edge(et, lw, lb, cw, cb):
    def body(x_ref, lw_ref, lb_ref, cw_ref, cb_ref, e_ref, ce_ref):
        e = _dot(x_ref[...], lw_ref[...]) + lb_ref[...]
        e_ref[...] = e
        ce_ref[...] = _dot(e, cw_ref[...]) + cb_ref[...]
    return pl.pallas_call(
        body,
        grid=(_EG,),
        in_specs=[
            pl.BlockSpec((_ER, 16), lambda i: (i, 0)),
            pl.BlockSpec((16, HID), lambda i: (0, 0)),
            pl.BlockSpec((1, HID), lambda i: (0, 0)),
            pl.BlockSpec((HID, HID), lambda i: (0, 0)),
            pl.BlockSpec((1, HID), lambda i: (0, 0)),
        ],
        out_specs=[
            pl.BlockSpec((_ER, HID), lambda i: (i, 0)),
            pl.BlockSpec((_ER, HID), lambda i: (i, 0)),
        ],
        out_shape=[
            jax.ShapeDtypeStruct((NE, HID), F32),
            jax.ShapeDtypeStruct((NE, HID), F32),
        ],
    )(et, lw, lb.reshape(1, HID), cw, cb.reshape(1, HID))


def _node_mm(h, lp):
    def body(h_ref, aw, ab, bw, bb, dw, dbias, ew, eb, ah_ref, dbt_ref, eht_ref):
        hh = h_ref[...]
        Ah = _dot(hh, aw[...]) + ab[...]
        Bh = _dot(hh, bw[...]) + bb[...]
        Dh = _dot(hh, dw[...]) + dbias[...]
        Eh = _dot(hh, ew[...]) + eb[...]
        ah_ref[...] = Ah
        dbt_ref[0] = jnp.concatenate([Dh[:, :HH], Bh[:, :HH]], axis=1)
        dbt_ref[1] = jnp.concatenate([Dh[:, HH:], Bh[:, HH:]], axis=1)
        eht_ref[0] = Eh[:, :HH]
        eht_ref[1] = Eh[:, HH:]
    return pl.pallas_call(
        body,
        out_shape=[
            jax.ShapeDtypeStruct((NN, HID), F32),
            jax.ShapeDtypeStruct((2, NN, HID), F32),
            jax.ShapeDtypeStruct((2, NN, HH), F32),
        ],
    )(h, lp['A_w'], lp['A_b'].reshape(1, HID),
      lp['B_w'], lp['B_b'].reshape(1, HID),
      lp['D_w'], lp['D_b'].reshape(1, HID),
      lp['E_w'], lp['E_b'].reshape(1, HID))


def _h_update(h_in, ah, acc, norm_n, bn_g, bn_b):
    def body(hin_ref, ah_ref, acc_ref, nn_ref, g_ref, b_ref, out_ref):
        num_f = jnp.concatenate([acc_ref[0, :NN, :HH], acc_ref[1, :NN, :HH]],
                                axis=1)
        den_f = jnp.concatenate([acc_ref[0, :NN, HH:], acc_ref[1, :NN, HH:]],
                                axis=1)
        hn = ah_ref[...] + num_f / (den_f + 1e-6)
        hn = hn * nn_ref[...]
        mu = jnp.mean(hn, axis=0, keepdims=True)
        var = jnp.mean((hn - mu) ** 2, axis=0, keepdims=True)
        hn = (hn - mu) * lax.rsqrt(var + 1e-5) * g_ref[...] + b_ref[...]
        out_ref[...] = hin_ref[...] + jnp.maximum(hn, 0.0)
    return pl.pallas_call(
        body, out_shape=jax.ShapeDtypeStruct((NN, HID), F32),
    )(h_in, ah, acc, norm_n, bn_g.reshape(1, HID), bn_b.reshape(1, HID))


def _e_stats(eraw, norm_e):
    def body(er_ref, ne_ref, out_ref):
        i = pl.program_id(0)
        y = er_ref[...] * ne_ref[...]
        @pl.when(i == 0)
        def _():
            out_ref[...] = jnp.zeros_like(out_ref)
        out_ref[0:1, :] += jnp.sum(y, axis=0, keepdims=True)
        out_ref[1:2, :] += jnp.sum(y * y, axis=0, keepdims=True)
    return pl.pallas_call(
        body,
        grid=(_EG,),
        in_specs=[
            pl.BlockSpec((_ER, HID), lambda i: (i, 0)),
            pl.BlockSpec((_ER, 1), lambda i: (i, 0)),
        ],
        out_specs=pl.BlockSpec((8, HID), lambda i: (0, 0)),
        out_shape=jax.ShapeDtypeStruct((8, HID), F32),
    )(eraw, norm_e)


def _e_apply(eraw, e_in, norm_e, st, bn_g, bn_b, cw, cb):
    def body(er_ref, ein_ref, ne_ref, st_ref, g_ref, b_ref, cw_ref, cb_ref,
             enew_ref, ce_ref):
        y = er_ref[...] * ne_ref[...]
        mu = st_ref[0:1, :] * (1.0 / NE)
        var = st_ref[1:2, :] * (1.0 / NE) - mu * mu
        z = (y - mu) * lax.rsqrt(var + 1e-5) * g_ref[...] + b_ref[...]
        z = jnp.maximum(z, 0.0)
        enew = ein_ref[...] + z
        enew_ref[...] = enew
        ce_ref[...] = _dot(enew, cw_ref[...]) + cb_ref[...]
    return pl.pallas_call(
        body,
        grid=(_EG,),
        in_specs=[
            pl.BlockSpec((_ER, HID), lambda i: (i, 0)),
            pl.BlockSpec((_ER, HID), lambda i: (i, 0)),
            pl.BlockSpec((_ER, 1), lambda i: (i, 0)),
            pl.BlockSpec((8, HID), lambda i: (0, 0)),
            pl.BlockSpec((1, HID), lambda i: (0, 0)),
            pl.BlockSpec((1, HID), lambda i: (0, 0)),
            pl.BlockSpec((HID, HID), lambda i: (0, 0)),
            pl.BlockSpec((1, HID), lambda i: (0, 0)),
        ],
        out_specs=[
            pl.BlockSpec((_ER, HID), lambda i: (i, 0)),
            pl.BlockSpec((_ER, HID), lambda i: (i, 0)),
        ],
        out_shape=[
            jax.ShapeDtypeStruct((NE, HID), F32),
            jax.ShapeDtypeStruct((NE, HID), F32),
        ],
    )(eraw, e_in, norm_e, st, bn_g.reshape(1, HID), bn_b.reshape(1, HID),
      cw, cb.reshape(1, HID))


_HR = 4096
_HG = NTRIP // _HR


def _head_mlp(s, o, fc1_w, fc1_b, bn1_g, bn1_b, out_w, out_b):
    def body1(s_ref, o_ref, w1s_ref, w1o_ref, b1_ref, f_ref, st_ref):
        i = pl.program_id(0)
        f = (_dot(s_ref[...], w1s_ref[...]) + _dot(o_ref[...], w1o_ref[...])
             + b1_ref[...])
        f_ref[...] = f
        @pl.when(i == 0)
        def _():
            st_ref[...] = jnp.zeros_like(st_ref)
        st_ref[0:1, :] += jnp.sum(f, axis=0, keepdims=True)
        st_ref[1:2, :] += jnp.sum(f * f, axis=0, keepdims=True)

    f, st = pl.pallas_call(
        body1,
        grid=(_HG,),
        in_specs=[
            pl.BlockSpec((_HR, HID), lambda i: (i, 0)),
            pl.BlockSpec((_HR, HID), lambda i: (i, 0)),
            pl.BlockSpec((HID, 200), lambda i: (0, 0)),
            pl.BlockSpec((HID, 200), lambda i: (0, 0)),
            pl.BlockSpec((1, 200), lambda i: (0, 0)),
        ],
        out_specs=[
            pl.BlockSpec((_HR, 200), lambda i: (i, 0)),
            pl.BlockSpec((8, 200), lambda i: (0, 0)),
        ],
        out_shape=[
            jax.ShapeDtypeStruct((NTRIP, 200), F32),
            jax.ShapeDtypeStruct((8, 200), F32),
        ],
    )(s, o, fc1_w[:HID], fc1_w[HID:], fc1_b.reshape(1, -1))

    def body2(f_ref, st_ref, g_ref, b_ref, ow_ref, ob_ref, out_ref):
        mu = st_ref[0:1, :] * (1.0 / NTRIP)
        var = st_ref[1:2, :] * (1.0 / NTRIP) - mu * mu
        z = (f_ref[...] - mu) * lax.rsqrt(var + 1e-5) * g_ref[...] + b_ref[...]
        z = jnp.maximum(z, 0.0)
        out_ref[...] = _dot(z, ow_ref[...]) + ob_ref[...]

    return pl.pallas_call(
        body2,
        grid=(_HG,),
        in_specs=[
            pl.BlockSpec((_HR, 200), lambda i: (i, 0)),
            pl.BlockSpec((8, 200), lambda i: (0, 0)),
            pl.BlockSpec((1, 200), lambda i: (0, 0)),
            pl.BlockSpec((1, 200), lambda i: (0, 0)),
            pl.BlockSpec((200, 1), lambda i: (0, 0)),
            pl.BlockSpec((1, 1), lambda i: (0, 0)),
        ],
        out_specs=pl.BlockSpec((_HR, 1), lambda i: (i, 0)),
        out_shape=jax.ShapeDtypeStruct((NTRIP, 1), F32),
    )(f, st, bn1_g.reshape(1, -1), bn1_b.reshape(1, -1), out_w,
      out_b.reshape(1, 1))


def kernel(node_id, edge_type, norm_n, norm_e, params, g, triplets):
    p = params
    t0 = triplets[:, 0]
    t2 = triplets[:, 2]

    h = _proj_node(node_id, p['lh_w'], p['lh_b'])
    l0 = p['layers'][0]
    e, ce = _proj_edge(edge_type, p['le_w'], p['le_b'], l0['C_w'], l0['C_b'])

    for li in range(len(p['layers'])):
        lp = p['layers'][li]
        ah, dbt, eht = _node_mm(h, lp)
        dbt_f = dbt.reshape(_NC * NN, HID)
        eht_f = eht.reshape(_NC * NN, HH)
        last = li == len(p['layers']) - 1
        if last:
            acc = _edge_pass_nw(ce, dbt_f, eht_f, g)
            if isinstance(acc, (tuple, list)):
                acc = acc[0]
            eraw = None
        else:
            acc, eraw = _edge_pass_w(ce, dbt_f, eht_f, g)
        h = _h_update(h, ah, acc.reshape(2, _NNP, HID),
                      norm_n, lp['bn_h_g'], lp['bn_h_b'])
        if not last:
            nlp = p['layers'][li + 1]
            st = _e_stats(eraw, norm_e)
            e, ce = _e_apply(eraw, e, norm_e, st, lp['bn_e_g'], lp['bn_e_b'],
                             nlp['C_w'], nlp['C_b'])

    s, o = _head_gather(h, t0, t2)
    out = _head_mlp(s, o, p['fc1_w'], p['fc1_b'], p['bn1_g'], p['bn1_b'],
                    p['out_w'], p['out_b'])
    return (h, out)
